# Initial kernel scaffold; baseline (speedup 1.0000x reference)
import functools

import jax
import jax.numpy as jnp
from jax import lax
from jax.experimental import pallas as pl
from jax.experimental.pallas import tpu as pltpu
from jax.experimental.pallas import tpu_sc as plsc

N = 100000
E = 1600000
G = 64
NC, NS = 2, 16
N_ACC = 100352
ROWS_PT = N_ACC // NS
KCH = 16
EP = 1638400
BM = 2048
STEPS = N_ACC // BM


def _make_sc_agg(w, core_split):
    tiles = NS if core_split else NC * NS
    ept = EP // tiles
    cpt = ept // 128
    nsup = cpt // KCH
    acc_shape = (N_ACC,) if w is None else (N_ACC, w)
    row_shape = (128,) if w is None else (128, w)
    out_rows = NC * N_ACC
    out_shape = (out_rows,) if w is None else (out_rows, w)
    mesh = plsc.VectorSubcoreMesh(core_axis_name="c", subcore_axis_name="s")

    @functools.partial(
        pl.kernel,
        mesh=mesh,
        out_type=jax.ShapeDtypeStruct(out_shape, jnp.float32),
        scratch_types=[
            pltpu.VMEM((KCH, 128), jnp.int32),
            pltpu.VMEM((KCH, 128), jnp.int32),
            pltpu.VMEM(row_shape, jnp.float32),
            pltpu.VMEM_SHARED(acc_shape, jnp.float32),
            pltpu.SemaphoreType.DMA,
        ],
    )
    def kern(tab, src2d, dst2d, zeros, out, src_v, dst_v, rows_v, acc, sem):
        c = lax.axis_index("c")
        s = lax.axis_index("s")
        pltpu.sync_copy(zeros.at[pl.ds(s * ROWS_PT, ROWS_PT)],
                        acc.at[pl.ds(s * ROWS_PT, ROWS_PT)])
        plsc.subcore_barrier()
        tile = s if core_split else s * NC + c
        dst_ch0 = tile * cpt
        src_ch0 = c * (EP // 128) + dst_ch0 if core_split else dst_ch0

        def sup(u, carry):
            pltpu.sync_copy(src2d.at[pl.ds(src_ch0 + u * KCH, KCH)], src_v)
            pltpu.sync_copy(dst2d.at[pl.ds(dst_ch0 + u * KCH, KCH)], dst_v)

            def inner(j, carry2):
                pltpu.async_copy(tab.at[src_v.at[j]], rows_v, sem).wait()
                pltpu.sync_copy(rows_v, acc.at[dst_v.at[j]], add=True)
                return carry2

            return lax.fori_loop(0, KCH, inner, carry)

        lax.fori_loop(0, nsup, sup, 0)
        plsc.subcore_barrier()
        pltpu.sync_copy(acc.at[pl.ds(s * ROWS_PT, ROWS_PT)],
                        out.at[pl.ds(c * N_ACC + s * ROWS_PT, ROWS_PT)])

    return kern


def _make_sc_deg():
    cpt = EP // (NC * NS * 128)
    nsup = cpt // KCH
    mesh = plsc.VectorSubcoreMesh(core_axis_name="c", subcore_axis_name="s")

    @functools.partial(
        pl.kernel,
        mesh=mesh,
        out_type=jax.ShapeDtypeStruct((NC * N_ACC,), jnp.float32),
        scratch_types=[
            pltpu.VMEM((KCH, 128), jnp.int32),
            pltpu.VMEM((128,), jnp.float32),
            pltpu.VMEM_SHARED((N_ACC,), jnp.float32),
        ],
    )
    def kern(dst2d, zeros, out, dst_v, ones_v, acc):
        c = lax.axis_index("c")
        s = lax.axis_index("s")
        for k in range(8):
            ones_v[pl.ds(16 * k, 16)] = jnp.ones((16,), jnp.float32)
        pltpu.sync_copy(zeros.at[pl.ds(s * ROWS_PT, ROWS_PT)],
                        acc.at[pl.ds(s * ROWS_PT, ROWS_PT)])
        plsc.subcore_barrier()
        tile = s * NC + c
        ch0 = tile * cpt

        def sup(u, carry):
            pltpu.sync_copy(dst2d.at[pl.ds(ch0 + u * KCH, KCH)], dst_v)

            def inner(j, carry2):
                pltpu.sync_copy(ones_v, acc.at[dst_v.at[j]], add=True)
                return carry2

            return lax.fori_loop(0, KCH, inner, carry)

        lax.fori_loop(0, nsup, sup, 0)
        plsc.subcore_barrier()
        pltpu.sync_copy(acc.at[pl.ds(s * ROWS_PT, ROWS_PT)],
                        out.at[pl.ds(c * N_ACC + s * ROWS_PT, ROWS_PT)])

    return kern


_sc_deg = _make_sc_deg()
_sc_w8 = _make_sc_agg(8, core_split=False)
_sc_w16 = _make_sc_agg(16, core_split=True)
_sc_w1 = _make_sc_agg(None, core_split=False)


def _row_spec(w):
    return pl.BlockSpec((BM, w), lambda i: (i, 0))


def _full_spec(shape):
    return pl.BlockSpec(shape, lambda i: tuple(0 for _ in shape))


def _tc1_body(d0, d1, x, s_o, g0_o):
    s = lax.rsqrt(d0[...] + d1[...] + 1.0)
    s_o[...] = s
    g0_o[...] = x[...] * s


def _tc1(d0, d1, x_p):
    return pl.pallas_call(
        _tc1_body,
        grid=(STEPS,),
        in_specs=[_row_spec(1), _row_spec(1), _row_spec(8)],
        out_specs=[_row_spec(1), _row_spec(8)],
        out_shape=[
            jax.ShapeDtypeStruct((N_ACC, 1), jnp.float32),
            jax.ShapeDtypeStruct((N_ACC, 8), jnp.float32),
        ],
    )(d0, d1, x_p)


def _tc2_body(q0, q1, g0, s, w1, b1, w2, g2_o):
    a1 = s[...] * (q0[...] + q1[...] + g0[...])
    h1 = lax.dot_general(a1, w1[...], (((1,), (1,)), ((), ())),
                         preferred_element_type=jnp.float32)
    h1 = jnp.maximum(h1 + b1[...], 0.0)
    p2 = lax.dot_general(h1, w2[...], (((1,), (1,)), ((), ())),
                         preferred_element_type=jnp.float32)
    g2_o[...] = s[...] * p2


def _tc2(q0, q1, g0, s_col, W1, b1r, W2):
    return pl.pallas_call(
        _tc2_body,
        grid=(STEPS,),
        in_specs=[_row_spec(8), _row_spec(8), _row_spec(8), _row_spec(1),
                  _full_spec((128, 8)), _full_spec((1, 128)),
                  _full_spec((32, 128))],
        out_specs=_row_spec(32),
        out_shape=jax.ShapeDtypeStruct((N_ACC, 32), jnp.float32),
    )(q0, q1, g0, s_col, W1, b1r, W2)


def _tc3_body(pc0, pc1, g2, s, w3, b2, g3_o):
    agg2 = jnp.concatenate([pc0[...], pc1[...]], axis=1)
    h2 = jnp.maximum(s[...] * (agg2 + g2[...]) + b2[...], 0.0)
    p3 = lax.dot_general(h2, w3[...], (((1,), (1,)), ((), ())),
                         preferred_element_type=jnp.float32)
    g3_o[...] = s[...] * p3


def _tc3(pc0, pc1, g2, s_col, W3, b2r):
    return pl.pallas_call(
        _tc3_body,
        grid=(STEPS,),
        in_specs=[_row_spec(16), _row_spec(16), _row_spec(32), _row_spec(1),
                  _full_spec((1, 32)), _full_spec((1, 32))],
        out_specs=_row_spec(1),
        out_shape=jax.ShapeDtypeStruct((N_ACC, 1), jnp.float32),
    )(pc0, pc1, g2, s_col, W3, b2r)


def _tc4_body(r0, r1, g3, s, b3, bat, out, acc):
    i = pl.program_id(0)
    h3 = s[...] * (r0[...] + r1[...] + g3[...]) + b3[...]
    gid = lax.broadcasted_iota(jnp.int32, (BM, G), 1)
    mask = (bat[...] == gid).astype(jnp.float32)
    hp = jnp.concatenate([h3, jnp.ones_like(h3)], axis=1)
    part = lax.dot_general(mask, hp, (((0,), (0,)), ((), ())),
                           preferred_element_type=jnp.float32)

    @pl.when(i == 0)
    def _():
        acc[...] = jnp.zeros_like(acc)

    acc[...] += part

    @pl.when(i == STEPS - 1)
    def _():
        sums = acc[:, 0:1]
        cnts = jnp.maximum(acc[:, 1:2], 1.0)
        out[...] = jax.nn.sigmoid(sums / cnts)


def _tc4(r0, r1, g3, s_col, b3r, batch_col):
    return pl.pallas_call(
        _tc4_body,
        grid=(STEPS,),
        in_specs=[_row_spec(1), _row_spec(1), _row_spec(1), _row_spec(1),
                  _full_spec((1, 1)), _row_spec(1)],
        out_specs=_full_spec((G, 1)),
        out_shape=jax.ShapeDtypeStruct((G, 1), jnp.float32),
        scratch_shapes=[pltpu.VMEM((G, 2), jnp.float32)],
    )(r0, r1, g3, s_col, b3r, batch_col)


def kernel(x, edge_index, batch, W1, b1, W2, b2, W3, b3):
    src = edge_index[0]
    dst = edge_index[1]
    pad = EP - E
    src_p = jnp.concatenate([src, jnp.zeros((pad,), jnp.int32)])
    dst_p = jnp.concatenate([dst, jnp.full((pad,), N, jnp.int32)])
    src2d = src_p.reshape(-1, 128)
    dst2d = dst_p.reshape(-1, 128)
    src2c = jnp.concatenate([src_p, src_p + N]).reshape(-1, 128)
    z1 = jnp.zeros((N_ACC,), jnp.float32)
    z8 = jnp.zeros((N_ACC, 8), jnp.float32)
    z16 = jnp.zeros((N_ACC, 16), jnp.float32)

    degp = _sc_deg(dst2d, z1)
    d0 = degp[:N_ACC, None]
    d1 = degp[N_ACC:, None]
    x_p = jnp.concatenate([x, jnp.zeros((N_ACC - N, 8), jnp.float32)])
    s_col, g0 = _tc1(d0, d1, x_p)

    q = _sc_w8(g0[:N], src2d, dst2d, z8)
    g2 = _tc2(q[:N_ACC], q[N_ACC:], g0, s_col, W1, b1[None, :], W2)

    g2tab = jnp.concatenate([g2[:N, :16], g2[:N, 16:]], axis=0)
    p16 = _sc_w16(g2tab, src2c, dst2d, z16)
    g3 = _tc3(p16[:N_ACC], p16[N_ACC:], g2, s_col, W3, b2[None, :])

    r = _sc_w1(g3[:N, 0], src2d, dst2d, z1)
    batch_p = jnp.concatenate([batch, jnp.full((N_ACC - N,), G, jnp.int32)])
    return _tc4(r[:N_ACC, None], r[N_ACC:, None], g3, s_col,
                b3[None, :], batch_p[:, None])


# speedup vs baseline: 19.0025x; 19.0025x over previous
import functools

import jax
import jax.numpy as jnp
from jax import lax
from jax.experimental import pallas as pl
from jax.experimental.pallas import tpu as pltpu
from jax.experimental.pallas import tpu_sc as plsc

N = 100000
E = 1600000
G = 64
NC, NS = 2, 16
N_ACC = 100352
ROWS_PT = N_ACC // NS
KCH = 16
EP = 1638400
BM = 2048
STEPS = N_ACC // BM


def _make_sc_agg(w, core_split):
    tiles = NS if core_split else NC * NS
    ept = EP // tiles
    cpt = ept // 128
    nsup = cpt // KCH
    acc_shape = (N_ACC,) if w is None else (N_ACC, w)
    row_shape = (128,) if w is None else (128, w)
    out_rows = NC * N_ACC
    out_shape = (out_rows,) if w is None else (out_rows, w)
    mesh = plsc.VectorSubcoreMesh(core_axis_name="c", subcore_axis_name="s")

    @functools.partial(
        pl.kernel,
        mesh=mesh,
        out_type=jax.ShapeDtypeStruct(out_shape, jnp.float32),
        compiler_params=pltpu.CompilerParams(use_tc_tiling_on_sc=False),
        scratch_types=[
            pltpu.VMEM((KCH, 128), jnp.int32),
            pltpu.VMEM((KCH, 128), jnp.int32),
            pltpu.VMEM(row_shape, jnp.float32),
            pltpu.VMEM_SHARED(acc_shape, jnp.float32),
            pltpu.SemaphoreType.DMA,
        ],
    )
    def kern(tab, src2d, dst2d, zeros, out, src_v, dst_v, rows_v, acc, sem):
        c = lax.axis_index("c")
        s = lax.axis_index("s")
        pltpu.sync_copy(zeros.at[pl.ds(s * ROWS_PT, ROWS_PT)],
                        acc.at[pl.ds(s * ROWS_PT, ROWS_PT)])
        plsc.subcore_barrier()
        tile = s if core_split else s * NC + c
        dst_ch0 = tile * cpt
        src_ch0 = c * (EP // 128) + dst_ch0 if core_split else dst_ch0

        def sup(u, carry):
            pltpu.sync_copy(src2d.at[pl.ds(src_ch0 + u * KCH, KCH)], src_v)
            pltpu.sync_copy(dst2d.at[pl.ds(dst_ch0 + u * KCH, KCH)], dst_v)

            def inner(j, carry2):
                pltpu.async_copy(tab.at[src_v.at[j]], rows_v, sem).wait()
                pltpu.sync_copy(rows_v, acc.at[dst_v.at[j]], add=True)
                return carry2

            return lax.fori_loop(0, KCH, inner, carry)

        lax.fori_loop(0, nsup, sup, 0)
        plsc.subcore_barrier()
        pltpu.sync_copy(acc.at[pl.ds(s * ROWS_PT, ROWS_PT)],
                        out.at[pl.ds(c * N_ACC + s * ROWS_PT, ROWS_PT)])

    return kern


def _make_sc_deg():
    cpt = EP // (NC * NS * 128)
    nsup = cpt // KCH
    mesh = plsc.VectorSubcoreMesh(core_axis_name="c", subcore_axis_name="s")

    @functools.partial(
        pl.kernel,
        mesh=mesh,
        out_type=jax.ShapeDtypeStruct((NC * N_ACC,), jnp.float32),
        compiler_params=pltpu.CompilerParams(use_tc_tiling_on_sc=False),
        scratch_types=[
            pltpu.VMEM((KCH, 128), jnp.int32),
            pltpu.VMEM((128,), jnp.float32),
            pltpu.VMEM_SHARED((N_ACC,), jnp.float32),
        ],
    )
    def kern(dst2d, zeros, out, dst_v, ones_v, acc):
        c = lax.axis_index("c")
        s = lax.axis_index("s")
        for k in range(8):
            ones_v[pl.ds(16 * k, 16)] = jnp.ones((16,), jnp.float32)
        pltpu.sync_copy(zeros.at[pl.ds(s * ROWS_PT, ROWS_PT)],
                        acc.at[pl.ds(s * ROWS_PT, ROWS_PT)])
        plsc.subcore_barrier()
        tile = s * NC + c
        ch0 = tile * cpt

        def sup(u, carry):
            pltpu.sync_copy(dst2d.at[pl.ds(ch0 + u * KCH, KCH)], dst_v)

            def inner(j, carry2):
                pltpu.sync_copy(ones_v, acc.at[dst_v.at[j]], add=True)
                return carry2

            return lax.fori_loop(0, KCH, inner, carry)

        lax.fori_loop(0, nsup, sup, 0)
        plsc.subcore_barrier()
        pltpu.sync_copy(acc.at[pl.ds(s * ROWS_PT, ROWS_PT)],
                        out.at[pl.ds(c * N_ACC + s * ROWS_PT, ROWS_PT)])

    return kern


_sc_deg = _make_sc_deg()
_sc_w8 = _make_sc_agg(8, core_split=False)
_sc_w16 = _make_sc_agg(16, core_split=True)
_sc_w1 = _make_sc_agg(None, core_split=False)


def _row_spec(w):
    return pl.BlockSpec((BM, w), lambda i: (i, 0))


def _full_spec(shape):
    return pl.BlockSpec(shape, lambda i: tuple(0 for _ in shape))


def _tc1_body(d0, d1, x, s_o, g0_o):
    s = lax.rsqrt(d0[...] + d1[...] + 1.0)
    s_o[...] = s
    g0_o[...] = x[...] * s


def _tc1(d0, d1, x_p):
    return pl.pallas_call(
        _tc1_body,
        grid=(STEPS,),
        in_specs=[_row_spec(1), _row_spec(1), _row_spec(8)],
        out_specs=[_row_spec(1), _row_spec(8)],
        out_shape=[
            jax.ShapeDtypeStruct((N_ACC, 1), jnp.float32),
            jax.ShapeDtypeStruct((N_ACC, 8), jnp.float32),
        ],
    )(d0, d1, x_p)


def _tc2_body(q0, q1, g0, s, w1, b1, w2, g2_o):
    a1 = s[...] * (q0[...] + q1[...] + g0[...])
    h1 = lax.dot_general(a1, w1[...], (((1,), (1,)), ((), ())),
                         preferred_element_type=jnp.float32)
    h1 = jnp.maximum(h1 + b1[...], 0.0)
    p2 = lax.dot_general(h1, w2[...], (((1,), (1,)), ((), ())),
                         preferred_element_type=jnp.float32)
    g2_o[...] = s[...] * p2


def _tc2(q0, q1, g0, s_col, W1, b1r, W2):
    return pl.pallas_call(
        _tc2_body,
        grid=(STEPS,),
        in_specs=[_row_spec(8), _row_spec(8), _row_spec(8), _row_spec(1),
                  _full_spec((128, 8)), _full_spec((1, 128)),
                  _full_spec((32, 128))],
        out_specs=_row_spec(32),
        out_shape=jax.ShapeDtypeStruct((N_ACC, 32), jnp.float32),
    )(q0, q1, g0, s_col, W1, b1r, W2)


def _tc3_body(pc0, pc1, g2, s, w3, b2, g3_o):
    agg2 = jnp.concatenate([pc0[...], pc1[...]], axis=1)
    h2 = jnp.maximum(s[...] * (agg2 + g2[...]) + b2[...], 0.0)
    p3 = lax.dot_general(h2, w3[...], (((1,), (1,)), ((), ())),
                         preferred_element_type=jnp.float32)
    g3_o[...] = s[...] * p3


def _tc3(pc0, pc1, g2, s_col, W3, b2r):
    return pl.pallas_call(
        _tc3_body,
        grid=(STEPS,),
        in_specs=[_row_spec(16), _row_spec(16), _row_spec(32), _row_spec(1),
                  _full_spec((1, 32)), _full_spec((1, 32))],
        out_specs=_row_spec(1),
        out_shape=jax.ShapeDtypeStruct((N_ACC, 1), jnp.float32),
    )(pc0, pc1, g2, s_col, W3, b2r)


def _tc4_body(r0, r1, g3, s, b3, bat, out, acc):
    i = pl.program_id(0)
    h3 = s[...] * (r0[...] + r1[...] + g3[...]) + b3[...]
    gid = lax.broadcasted_iota(jnp.int32, (BM, G), 1)
    mask = (bat[...] == gid).astype(jnp.float32)
    hp = jnp.concatenate([h3, jnp.ones_like(h3)], axis=1)
    part = lax.dot_general(mask, hp, (((0,), (0,)), ((), ())),
                           preferred_element_type=jnp.float32)

    @pl.when(i == 0)
    def _():
        acc[...] = jnp.zeros_like(acc)

    acc[...] += part

    @pl.when(i == STEPS - 1)
    def _():
        sums = acc[:, 0:1]
        cnts = jnp.maximum(acc[:, 1:2], 1.0)
        out[...] = jax.nn.sigmoid(sums / cnts)


def _tc4(r0, r1, g3, s_col, b3r, batch_col):
    return pl.pallas_call(
        _tc4_body,
        grid=(STEPS,),
        in_specs=[_row_spec(1), _row_spec(1), _row_spec(1), _row_spec(1),
                  _full_spec((1, 1)), _row_spec(1)],
        out_specs=_full_spec((G, 1)),
        out_shape=jax.ShapeDtypeStruct((G, 1), jnp.float32),
        scratch_shapes=[pltpu.VMEM((G, 2), jnp.float32)],
    )(r0, r1, g3, s_col, b3r, batch_col)


def kernel(x, edge_index, batch, W1, b1, W2, b2, W3, b3):
    src = edge_index[0]
    dst = edge_index[1]
    pad = EP - E
    src_p = jnp.concatenate([src, jnp.zeros((pad,), jnp.int32)])
    dst_p = jnp.concatenate([dst, jnp.full((pad,), N, jnp.int32)])
    src2d = src_p.reshape(-1, 128)
    dst2d = dst_p.reshape(-1, 128)
    src2c = jnp.concatenate([src_p, src_p + N]).reshape(-1, 128)
    z1 = jnp.zeros((N_ACC,), jnp.float32)
    z8 = jnp.zeros((N_ACC, 8), jnp.float32)
    z16 = jnp.zeros((N_ACC, 16), jnp.float32)

    degp = _sc_deg(dst2d, z1)
    d0 = degp[:N_ACC, None]
    d1 = degp[N_ACC:, None]
    x_p = jnp.concatenate([x, jnp.zeros((N_ACC - N, 8), jnp.float32)])
    s_col, g0 = _tc1(d0, d1, x_p)

    q = _sc_w8(g0[:N], src2d, dst2d, z8)
    g2 = _tc2(q[:N_ACC], q[N_ACC:], g0, s_col, W1, b1[None, :], W2)

    g2tab = jnp.concatenate([g2[:N, :16], g2[:N, 16:]], axis=0)
    p16 = _sc_w16(g2tab, src2c, dst2d, z16)
    g3 = _tc3(p16[:N_ACC], p16[N_ACC:], g2, s_col, W3, b2[None, :])

    r = _sc_w1(g3[:N, 0], src2d, dst2d, z1)
    batch_p = jnp.concatenate([batch, jnp.full((N_ACC - N,), G, jnp.int32)])
    return _tc4(r[:N_ACC, None], r[N_ACC:, None], g3, s_col,
                b3[None, :], batch_p[:, None])


# speedup vs baseline: 26.8542x; 1.4132x over previous
import functools

import jax
import jax.numpy as jnp
from jax import lax
from jax.experimental import pallas as pl
from jax.experimental.pallas import tpu as pltpu
from jax.experimental.pallas import tpu_sc as plsc

N = 100000
E = 1600000
G = 64
NC, NS = 2, 16
N_ACC = 100352
ROWS_PT = N_ACC // NS
KCH = 8
EP = 1638400
BM = 2048
STEPS = N_ACC // BM


def _make_sc_agg(w, core_split, kch=KCH):
    tiles = NS if core_split else NC * NS
    ept = EP // tiles
    cpt = ept // 128
    nblk = cpt // kch
    assert nblk % 2 == 0
    acc_shape = (N_ACC,) if w is None else (N_ACC, w)
    rows_shape = (2, KCH * 128) if w is None else (2, kch * 128, w)
    out_rows = NC * N_ACC
    out_shape = (out_rows,) if w is None else (out_rows, w)
    mesh = plsc.VectorSubcoreMesh(core_axis_name="c", subcore_axis_name="s")

    @functools.partial(
        pl.kernel,
        mesh=mesh,
        out_type=jax.ShapeDtypeStruct(out_shape, jnp.float32),
        compiler_params=pltpu.CompilerParams(use_tc_tiling_on_sc=False),
        scratch_types=[
            pltpu.VMEM((2, kch, 128), jnp.int32),
            pltpu.VMEM((2, kch, 128), jnp.int32),
            pltpu.VMEM(rows_shape, jnp.float32),
            pltpu.VMEM_SHARED(acc_shape, jnp.float32),
            pltpu.SemaphoreType.DMA,
            pltpu.SemaphoreType.DMA,
            pltpu.SemaphoreType.DMA,
            pltpu.SemaphoreType.DMA,
        ],
    )
    def kern(tab, src2d, dst2d, zeros, out, src_i, dst_i, rows, acc,
             sg0, sg1, ss0, ss1):
        c = lax.axis_index("c")
        s = lax.axis_index("s")
        pltpu.sync_copy(zeros.at[pl.ds(s * ROWS_PT, ROWS_PT)],
                        acc.at[pl.ds(s * ROWS_PT, ROWS_PT)])
        plsc.subcore_barrier()
        tile = s if core_split else s * NC + c
        dst_ch0 = tile * cpt
        src_ch0 = c * (EP // 128) + dst_ch0 if core_split else dst_ch0
        sg = (sg0, sg1)
        ss = (ss0, ss1)

        def row_slot(b, j):
            if w is None:
                return rows.at[b, pl.ds(j * 128, 128)]
            return rows.at[b, pl.ds(j * 128, 128), :]

        def load_idx(k, b):
            pltpu.sync_copy(src2d.at[pl.ds(src_ch0 + k * kch, kch)],
                            src_i.at[b])
            pltpu.sync_copy(dst2d.at[pl.ds(dst_ch0 + k * kch, kch)],
                            dst_i.at[b])

        def start_gathers(b):
            for j in range(kch):
                pltpu.async_copy(tab.at[src_i.at[b, j]], row_slot(b, j),
                                 sg[b])

        def drain_gathers(b):
            for j in range(kch):
                pltpu.make_async_copy(tab.at[src_i.at[b, j]],
                                      row_slot(b, j), sg[b]).wait()

        def start_scatters(b):
            for j in range(kch):
                pltpu.async_copy(row_slot(b, j), acc.at[dst_i.at[b, j]],
                                 ss[b], add=True)

        def drain_scatters(b):
            for j in range(kch):
                pltpu.make_async_copy(row_slot(b, j),
                                      acc.at[dst_i.at[b, j]], ss[b]).wait()

        load_idx(0, 0)
        start_gathers(0)

        def phase(k, b):
            nb = 1 - b

            @pl.when(k >= 1)
            def _():
                drain_scatters(nb)

            @pl.when(k + 1 < nblk)
            def _():
                load_idx(k + 1, nb)
                start_gathers(nb)

            drain_gathers(b)
            start_scatters(b)

        def body(uu, carry):
            phase(2 * uu, 0)
            phase(2 * uu + 1, 1)
            return carry

        lax.fori_loop(0, nblk // 2, body, 0)
        drain_scatters(1)
        plsc.subcore_barrier()
        pltpu.sync_copy(acc.at[pl.ds(s * ROWS_PT, ROWS_PT)],
                        out.at[pl.ds(c * N_ACC + s * ROWS_PT, ROWS_PT)])

    return kern


def _make_sc_deg():
    cpt = EP // (NC * NS * 128)
    nblk = cpt // KCH
    assert nblk % 2 == 0
    mesh = plsc.VectorSubcoreMesh(core_axis_name="c", subcore_axis_name="s")

    @functools.partial(
        pl.kernel,
        mesh=mesh,
        out_type=jax.ShapeDtypeStruct((NC * N_ACC,), jnp.float32),
        compiler_params=pltpu.CompilerParams(use_tc_tiling_on_sc=False),
        scratch_types=[
            pltpu.VMEM((2, KCH, 128), jnp.int32),
            pltpu.VMEM((128,), jnp.float32),
            pltpu.VMEM_SHARED((N_ACC,), jnp.float32),
            pltpu.SemaphoreType.DMA,
            pltpu.SemaphoreType.DMA,
        ],
    )
    def kern(dst2d, zeros, out, dst_i, ones_v, acc, ss0, ss1):
        c = lax.axis_index("c")
        s = lax.axis_index("s")
        for k in range(8):
            ones_v[pl.ds(16 * k, 16)] = jnp.ones((16,), jnp.float32)
        pltpu.sync_copy(zeros.at[pl.ds(s * ROWS_PT, ROWS_PT)],
                        acc.at[pl.ds(s * ROWS_PT, ROWS_PT)])
        plsc.subcore_barrier()
        tile = s * NC + c
        ch0 = tile * cpt
        ss = (ss0, ss1)

        def start_scatters(b):
            for j in range(KCH):
                pltpu.async_copy(ones_v, acc.at[dst_i.at[b, j]], ss[b],
                                 add=True)

        def drain_scatters(b):
            for j in range(KCH):
                pltpu.make_async_copy(ones_v, acc.at[dst_i.at[b, j]],
                                      ss[b]).wait()

        pltpu.sync_copy(dst2d.at[pl.ds(ch0, KCH)], dst_i.at[0])

        def phase(k, b):
            nb = 1 - b

            @pl.when(k >= 1)
            def _():
                drain_scatters(nb)

            @pl.when(k + 1 < nblk)
            def _():
                pltpu.sync_copy(dst2d.at[pl.ds(ch0 + (k + 1) * KCH, KCH)],
                                dst_i.at[nb])

            start_scatters(b)

        def body(uu, carry):
            phase(2 * uu, 0)
            phase(2 * uu + 1, 1)
            return carry

        lax.fori_loop(0, nblk // 2, body, 0)
        drain_scatters(1)
        plsc.subcore_barrier()
        pltpu.sync_copy(acc.at[pl.ds(s * ROWS_PT, ROWS_PT)],
                        out.at[pl.ds(c * N_ACC + s * ROWS_PT, ROWS_PT)])

    return kern


_sc_deg = _make_sc_deg()
_sc_w8 = _make_sc_agg(8, core_split=False)
_sc_w16 = _make_sc_agg(16, core_split=True, kch=4)
_sc_w1 = _make_sc_agg(None, core_split=False)


def _row_spec(w):
    return pl.BlockSpec((BM, w), lambda i: (i, 0))


def _full_spec(shape):
    return pl.BlockSpec(shape, lambda i: tuple(0 for _ in shape))


def _tc1_body(d0, d1, x, s_o, g0_o):
    s = lax.rsqrt(d0[...] + d1[...] + 1.0)
    s_o[...] = s
    g0_o[...] = x[...] * s


def _tc1(d0, d1, x_p):
    return pl.pallas_call(
        _tc1_body,
        grid=(STEPS,),
        in_specs=[_row_spec(1), _row_spec(1), _row_spec(8)],
        out_specs=[_row_spec(1), _row_spec(8)],
        out_shape=[
            jax.ShapeDtypeStruct((N_ACC, 1), jnp.float32),
            jax.ShapeDtypeStruct((N_ACC, 8), jnp.float32),
        ],
    )(d0, d1, x_p)


def _tc2_body(q0, q1, g0, s, w1, b1, w2, g2_o):
    a1 = s[...] * (q0[...] + q1[...] + g0[...])
    h1 = lax.dot_general(a1, w1[...], (((1,), (1,)), ((), ())),
                         preferred_element_type=jnp.float32)
    h1 = jnp.maximum(h1 + b1[...], 0.0)
    p2 = lax.dot_general(h1, w2[...], (((1,), (1,)), ((), ())),
                         preferred_element_type=jnp.float32)
    g2_o[...] = s[...] * p2


def _tc2(q0, q1, g0, s_col, W1, b1r, W2):
    return pl.pallas_call(
        _tc2_body,
        grid=(STEPS,),
        in_specs=[_row_spec(8), _row_spec(8), _row_spec(8), _row_spec(1),
                  _full_spec((128, 8)), _full_spec((1, 128)),
                  _full_spec((32, 128))],
        out_specs=_row_spec(32),
        out_shape=jax.ShapeDtypeStruct((N_ACC, 32), jnp.float32),
    )(q0, q1, g0, s_col, W1, b1r, W2)


def _tc3_body(pc0, pc1, g2, s, w3, b2, g3_o):
    agg2 = jnp.concatenate([pc0[...], pc1[...]], axis=1)
    h2 = jnp.maximum(s[...] * (agg2 + g2[...]) + b2[...], 0.0)
    p3 = lax.dot_general(h2, w3[...], (((1,), (1,)), ((), ())),
                         preferred_element_type=jnp.float32)
    g3_o[...] = s[...] * p3


def _tc3(pc0, pc1, g2, s_col, W3, b2r):
    return pl.pallas_call(
        _tc3_body,
        grid=(STEPS,),
        in_specs=[_row_spec(16), _row_spec(16), _row_spec(32), _row_spec(1),
                  _full_spec((1, 32)), _full_spec((1, 32))],
        out_specs=_row_spec(1),
        out_shape=jax.ShapeDtypeStruct((N_ACC, 1), jnp.float32),
    )(pc0, pc1, g2, s_col, W3, b2r)


def _tc4_body(r0, r1, g3, s, b3, bat, out, acc):
    i = pl.program_id(0)
    h3 = s[...] * (r0[...] + r1[...] + g3[...]) + b3[...]
    gid = lax.broadcasted_iota(jnp.int32, (BM, G), 1)
    mask = (bat[...] == gid).astype(jnp.float32)
    hp = jnp.concatenate([h3, jnp.ones_like(h3)], axis=1)
    part = lax.dot_general(mask, hp, (((0,), (0,)), ((), ())),
                           preferred_element_type=jnp.float32)

    @pl.when(i == 0)
    def _():
        acc[...] = jnp.zeros_like(acc)

    acc[...] += part

    @pl.when(i == STEPS - 1)
    def _():
        sums = acc[:, 0:1]
        cnts = jnp.maximum(acc[:, 1:2], 1.0)
        out[...] = jax.nn.sigmoid(sums / cnts)


def _tc4(r0, r1, g3, s_col, b3r, batch_col):
    return pl.pallas_call(
        _tc4_body,
        grid=(STEPS,),
        in_specs=[_row_spec(1), _row_spec(1), _row_spec(1), _row_spec(1),
                  _full_spec((1, 1)), _row_spec(1)],
        out_specs=_full_spec((G, 1)),
        out_shape=jax.ShapeDtypeStruct((G, 1), jnp.float32),
        scratch_shapes=[pltpu.VMEM((G, 2), jnp.float32)],
    )(r0, r1, g3, s_col, b3r, batch_col)


def kernel(x, edge_index, batch, W1, b1, W2, b2, W3, b3):
    src = edge_index[0]
    dst = edge_index[1]
    pad = EP - E
    src_p = jnp.concatenate([src, jnp.zeros((pad,), jnp.int32)])
    dst_p = jnp.concatenate([dst, jnp.full((pad,), N, jnp.int32)])
    src2d = src_p.reshape(-1, 128)
    dst2d = dst_p.reshape(-1, 128)
    src2c = jnp.concatenate([src_p, src_p + N]).reshape(-1, 128)
    z1 = jnp.zeros((N_ACC,), jnp.float32)
    z8 = jnp.zeros((N_ACC, 8), jnp.float32)
    z16 = jnp.zeros((N_ACC, 16), jnp.float32)

    degp = _sc_deg(dst2d, z1)
    d0 = degp[:N_ACC, None]
    d1 = degp[N_ACC:, None]
    x_p = jnp.concatenate([x, jnp.zeros((N_ACC - N, 8), jnp.float32)])
    s_col, g0 = _tc1(d0, d1, x_p)

    q = _sc_w8(g0[:N], src2d, dst2d, z8)
    g2 = _tc2(q[:N_ACC], q[N_ACC:], g0, s_col, W1, b1[None, :], W2)

    g2tab = jnp.concatenate([g2[:N, :16], g2[:N, 16:]], axis=0)
    p16 = _sc_w16(g2tab, src2c, dst2d, z16)
    g3 = _tc3(p16[:N_ACC], p16[N_ACC:], g2, s_col, W3, b2[None, :])

    r = _sc_w1(g3[:N, 0], src2d, dst2d, z1)
    batch_p = jnp.concatenate([batch, jnp.full((N_ACC - N,), G, jnp.int32)])
    return _tc4(r[:N_ACC, None], r[N_ACC:, None], g3, s_col,
                b3[None, :], batch_p[:, None])


# speedup vs baseline: 39.6956x; 1.4782x over previous
import functools

import jax
import jax.numpy as jnp
from jax import lax
from jax.experimental import pallas as pl
from jax.experimental.pallas import tpu as pltpu
from jax.experimental.pallas import tpu_sc as plsc

N = 100000
E = 1600000
G = 64
NC, NS = 2, 16
N_ACC = 100352
ROWS_PT = N_ACC // NS
KCH = 8
EP = 1638400
EPS = 1769472
BM = 2048
STEPS = N_ACC // BM
RS = N_ACC // 128


def _make_sc_agg(w, core_split, ep, kch=KCH):
    tiles = NS if core_split else NC * NS
    ept = ep // tiles
    cpt = ept // 128
    nblk = cpt // kch
    assert nblk % 2 == 0
    acc_shape = (N_ACC,) if w is None else (N_ACC, w)
    rows_shape = (2, kch * 128) if w is None else (2, kch * 128, w)
    out_rows = NC * N_ACC
    out_shape = (out_rows,) if w is None else (out_rows, w)
    mesh = plsc.VectorSubcoreMesh(core_axis_name="c", subcore_axis_name="s")

    @functools.partial(
        pl.kernel,
        mesh=mesh,
        out_type=jax.ShapeDtypeStruct(out_shape, jnp.float32),
        compiler_params=pltpu.CompilerParams(use_tc_tiling_on_sc=False),
        scratch_types=[
            pltpu.VMEM((2, kch, 128), jnp.int32),
            pltpu.VMEM((2, kch, 128), jnp.int32),
            pltpu.VMEM(rows_shape, jnp.float32),
            pltpu.VMEM_SHARED(acc_shape, jnp.float32),
            pltpu.SemaphoreType.DMA,
            pltpu.SemaphoreType.DMA,
            pltpu.SemaphoreType.DMA,
            pltpu.SemaphoreType.DMA,
        ],
    )
    def kern(tab, src2d, dst2d, zeros, out, src_i, dst_i, rows, acc,
             sg0, sg1, ss0, ss1):
        c = lax.axis_index("c")
        s = lax.axis_index("s")
        pltpu.sync_copy(zeros.at[pl.ds(s * ROWS_PT, ROWS_PT)],
                        acc.at[pl.ds(s * ROWS_PT, ROWS_PT)])
        plsc.subcore_barrier()
        tile = s if core_split else s * NC + c
        dst_ch0 = tile * cpt
        src_ch0 = c * (ep // 128) + dst_ch0 if core_split else dst_ch0
        sg = (sg0, sg1)
        ss = (ss0, ss1)

        def row_slot(b, j):
            if w is None:
                return rows.at[b, pl.ds(j * 128, 128)]
            return rows.at[b, pl.ds(j * 128, 128), :]

        def load_idx(k, b):
            pltpu.sync_copy(src2d.at[pl.ds(src_ch0 + k * kch, kch)],
                            src_i.at[b])
            pltpu.sync_copy(dst2d.at[pl.ds(dst_ch0 + k * kch, kch)],
                            dst_i.at[b])

        def start_gathers(b):
            for j in range(kch):
                pltpu.async_copy(tab.at[src_i.at[b, j]], row_slot(b, j),
                                 sg[b])

        def drain_gathers(b):
            for j in range(kch):
                pltpu.make_async_copy(tab.at[src_i.at[b, j]],
                                      row_slot(b, j), sg[b]).wait()

        def start_scatters(b):
            for j in range(kch):
                pltpu.async_copy(row_slot(b, j), acc.at[dst_i.at[b, j]],
                                 ss[b], add=True)

        def drain_scatters(b):
            for j in range(kch):
                pltpu.make_async_copy(row_slot(b, j),
                                      acc.at[dst_i.at[b, j]], ss[b]).wait()

        load_idx(0, 0)
        start_gathers(0)

        def phase(k, b):
            nb = 1 - b

            @pl.when(k >= 1)
            def _():
                drain_scatters(nb)

            @pl.when(k + 1 < nblk)
            def _():
                load_idx(k + 1, nb)
                start_gathers(nb)

            drain_gathers(b)
            start_scatters(b)

        def body(uu, carry):
            phase(2 * uu, 0)
            phase(2 * uu + 1, 1)
            return carry

        lax.fori_loop(0, nblk // 2, body, 0)
        drain_scatters(1)
        plsc.subcore_barrier()
        pltpu.sync_copy(acc.at[pl.ds(s * ROWS_PT, ROWS_PT)],
                        out.at[pl.ds(c * N_ACC + s * ROWS_PT, ROWS_PT)])

    return kern


def _make_sc_deg():
    cpt = EP // (NC * NS * 128)
    nblk = cpt // KCH
    assert nblk % 2 == 0
    mesh = plsc.VectorSubcoreMesh(core_axis_name="c", subcore_axis_name="s")

    @functools.partial(
        pl.kernel,
        mesh=mesh,
        out_type=jax.ShapeDtypeStruct((NC * N_ACC,), jnp.float32),
        compiler_params=pltpu.CompilerParams(use_tc_tiling_on_sc=False),
        scratch_types=[
            pltpu.VMEM((2, KCH, 128), jnp.int32),
            pltpu.VMEM((128,), jnp.float32),
            pltpu.VMEM_SHARED((N_ACC,), jnp.float32),
            pltpu.SemaphoreType.DMA,
            pltpu.SemaphoreType.DMA,
        ],
    )
    def kern(dst2d, zeros, out, dst_i, ones_v, acc, ss0, ss1):
        c = lax.axis_index("c")
        s = lax.axis_index("s")
        for k in range(8):
            ones_v[pl.ds(16 * k, 16)] = jnp.ones((16,), jnp.float32)
        pltpu.sync_copy(zeros.at[pl.ds(s * ROWS_PT, ROWS_PT)],
                        acc.at[pl.ds(s * ROWS_PT, ROWS_PT)])
        plsc.subcore_barrier()
        tile = s * NC + c
        ch0 = tile * cpt
        ss = (ss0, ss1)

        def start_scatters(b):
            for j in range(KCH):
                pltpu.async_copy(ones_v, acc.at[dst_i.at[b, j]], ss[b],
                                 add=True)

        def drain_scatters(b):
            for j in range(KCH):
                pltpu.make_async_copy(ones_v, acc.at[dst_i.at[b, j]],
                                      ss[b]).wait()

        pltpu.sync_copy(dst2d.at[pl.ds(ch0, KCH)], dst_i.at[0])

        def phase(k, b):
            nb = 1 - b

            @pl.when(k >= 1)
            def _():
                drain_scatters(nb)

            @pl.when(k + 1 < nblk)
            def _():
                pltpu.sync_copy(dst2d.at[pl.ds(ch0 + (k + 1) * KCH, KCH)],
                                dst_i.at[nb])

            start_scatters(b)

        def body(uu, carry):
            phase(2 * uu, 0)
            phase(2 * uu + 1, 1)
            return carry

        lax.fori_loop(0, nblk // 2, body, 0)
        drain_scatters(1)
        plsc.subcore_barrier()
        pltpu.sync_copy(acc.at[pl.ds(s * ROWS_PT, ROWS_PT)],
                        out.at[pl.ds(c * N_ACC + s * ROWS_PT, ROWS_PT)])

    return kern


_sc_deg = _make_sc_deg()
_sc_w8 = _make_sc_agg(8, core_split=False, ep=EP)
_sc_w16 = _make_sc_agg(16, core_split=True, ep=EP, kch=4)
_sc_w1 = _make_sc_agg(None, core_split=False, ep=EPS)


def _fs(rows, off=0):
    return pl.BlockSpec((rows, 128), lambda i, o=off: (i + o, 0))


def _full(shape):
    return pl.BlockSpec(shape, lambda i: tuple(0 for _ in shape))


def _tc1_body(d0, d1, x, k8, s_o, g0_o):
    s = lax.rsqrt(d0[...] + d1[...] + 1.0)
    s_o[...] = s
    v = jnp.reshape(s, (128, 16))
    srep8 = lax.dot_general(v, k8[...], (((1,), (0,)), ((), ())),
                            preferred_element_type=jnp.float32)
    g0_o[...] = x[...] * srep8


def _tc1(degf, x_flat, k8):
    return pl.pallas_call(
        _tc1_body,
        grid=(STEPS,),
        in_specs=[_fs(16), _fs(16, STEPS), _fs(128), _full((16, 128))],
        out_specs=[_fs(16), _fs(128)],
        out_shape=[
            jax.ShapeDtypeStruct((RS, 128), jnp.float32),
            jax.ShapeDtypeStruct((8 * RS, 128), jnp.float32),
        ],
    )(degf, degf, x_flat, k8)


def _tc2_body(q0, q1, g0, s, k8, k16, bd1, b1w, bd2lo, bd2hi, lo_o, hi_o):
    v = jnp.reshape(s[...], (128, 16))
    srep8 = lax.dot_general(v, k8[...], (((1,), (0,)), ((), ())),
                            preferred_element_type=jnp.float32)
    a1f = srep8 * (q0[...] + q1[...] + g0[...])
    h1w = lax.dot_general(a1f, bd1[...], (((1,), (0,)), ((), ())),
                          preferred_element_type=jnp.float32)
    h1w = jnp.maximum(h1w + b1w[...], 0.0)
    srepw16 = lax.dot_general(v, k16[...], (((1,), (0,)), ((), ())),
                              preferred_element_type=jnp.float32)
    p2lo = lax.dot_general(h1w, bd2lo[...], (((1,), (0,)), ((), ())),
                           preferred_element_type=jnp.float32)
    p2hi = lax.dot_general(h1w, bd2hi[...], (((1,), (0,)), ((), ())),
                           preferred_element_type=jnp.float32)
    lo_o[...] = jnp.reshape(srepw16 * p2lo, (256, 128))
    hi_o[...] = jnp.reshape(srepw16 * p2hi, (256, 128))


def _tc2(qf, g0_flat, s1, k8, k16, bd1, b1w, bd2lo, bd2hi):
    return pl.pallas_call(
        _tc2_body,
        grid=(STEPS,),
        in_specs=[_fs(128), _fs(128, STEPS), _fs(128), _fs(16),
                  _full((16, 128)), _full((16, 256)), _full((128, 2048)),
                  _full((1, 2048)), _full((2048, 256)), _full((2048, 256))],
        out_specs=[_fs(256), _fs(256)],
        out_shape=[
            jax.ShapeDtypeStruct((16 * RS, 128), jnp.float32),
            jax.ShapeDtypeStruct((16 * RS, 128), jnp.float32),
        ],
    )(qf, qf, g0_flat, s1, k8, k16, bd1, b1w, bd2lo, bd2hi)


def _tc3_body(pc0, pc1, glo, ghi, s, k16, m16lo, m16hi, b2lo, b2hi, g3_o):
    v = jnp.reshape(s[...], (128, 16))
    srepw16 = lax.dot_general(v, k16[...], (((1,), (0,)), ((), ())),
                              preferred_element_type=jnp.float32)
    srep16 = jnp.reshape(srepw16, (256, 128))
    h2lo = jnp.maximum(srep16 * (pc0[...] + glo[...]) + b2lo[...], 0.0)
    h2hi = jnp.maximum(srep16 * (pc1[...] + ghi[...]) + b2hi[...], 0.0)
    p3rep = (lax.dot_general(h2lo, m16lo[...], (((1,), (0,)), ((), ())),
                             preferred_element_type=jnp.float32)
             + lax.dot_general(h2hi, m16hi[...], (((1,), (0,)), ((), ())),
                               preferred_element_type=jnp.float32))
    g3_o[...] = srep16 * p3rep


def _tc3(p16f, glo_f, ghi_f, s1, k16, m16lo, m16hi, b2lo, b2hi):
    return pl.pallas_call(
        _tc3_body,
        grid=(STEPS,),
        in_specs=[_fs(256), _fs(256, STEPS), _fs(256), _fs(256), _fs(16),
                  _full((16, 256)), _full((128, 128)), _full((128, 128)),
                  _full((1, 128)), _full((1, 128))],
        out_specs=_fs(256),
        out_shape=jax.ShapeDtypeStruct((16 * RS, 128), jnp.float32),
    )(p16f, p16f, glo_f, ghi_f, s1, k16, m16lo, m16hi, b2lo, b2hi)


def _tc4_body(r0, r1, s, b3, bat, out, acc):
    i = pl.program_id(0)
    h3 = s[...] * (r0[...] + r1[...]) + b3[...]
    h3row = jnp.reshape(h3, (1, BM))
    brow = jnp.reshape(bat[...], (1, BM))
    gid = lax.broadcasted_iota(jnp.int32, (G, BM), 0)
    mask = (brow == gid).astype(jnp.float32)
    part_s = jnp.sum(mask * h3row, axis=1, keepdims=True)
    part_c = jnp.sum(mask, axis=1, keepdims=True)
    part = jnp.concatenate([part_s, part_c], axis=1)

    @pl.when(i == 0)
    def _():
        acc[...] = jnp.zeros_like(acc)

    acc[...] += part

    @pl.when(i == STEPS - 1)
    def _():
        sums = acc[:, 0:1]
        cnts = jnp.maximum(acc[:, 1:2], 1.0)
        out[...] = jax.nn.sigmoid(sums / cnts)


def _tc4(rf, s1, b3r, batchf):
    return pl.pallas_call(
        _tc4_body,
        grid=(STEPS,),
        in_specs=[_fs(16), _fs(16, STEPS), _fs(16), _full((1, 1)), _fs(16)],
        out_specs=_full((G, 1)),
        out_shape=jax.ShapeDtypeStruct((G, 1), jnp.float32),
        scratch_shapes=[pltpu.VMEM((G, 2), jnp.float32)],
    )(rf, rf, s1, b3r, batchf)


def kernel(x, edge_index, batch, W1, b1, W2, b2, W3, b3):
    src = edge_index[0]
    dst = edge_index[1]
    i16 = jnp.eye(16, dtype=jnp.float32)
    src_p = jnp.concatenate([src, jnp.zeros((EP - E,), jnp.int32)])
    dst_p = jnp.concatenate([dst, jnp.full((EP - E,), N, jnp.int32)])
    src2d = src_p.reshape(-1, 128)
    dst2d = dst_p.reshape(-1, 128)
    src2c = jnp.concatenate([src_p, src_p + N_ACC]).reshape(-1, 128)
    loop = jnp.arange(N_ACC, dtype=jnp.int32)
    srcw1 = jnp.concatenate(
        [src * 16, loop * 16,
         jnp.zeros((EPS - E - N_ACC,), jnp.int32)]).reshape(-1, 128)
    dstw1 = jnp.concatenate(
        [dst, loop, jnp.full((EPS - E - N_ACC,), N, jnp.int32)]
    ).reshape(-1, 128)
    z1 = jnp.zeros((N_ACC,), jnp.float32)
    z8 = jnp.zeros((N_ACC, 8), jnp.float32)
    z16 = jnp.zeros((N_ACC, 16), jnp.float32)
    k8 = jnp.repeat(i16, 8, axis=1)
    k16 = jnp.repeat(i16, 16, axis=1)
    bd1 = jnp.kron(i16, W1.T)
    b1w = jnp.tile(b1, 16)[None, :]
    bd2lo = jnp.kron(i16, W2.T[:, :16])
    bd2hi = jnp.kron(i16, W2.T[:, 16:])
    i8 = jnp.eye(8, dtype=jnp.float32)
    m16lo = jnp.kron(i8, jnp.outer(W3[0, :16], jnp.ones(16, jnp.float32)))
    m16hi = jnp.kron(i8, jnp.outer(W3[0, 16:], jnp.ones(16, jnp.float32)))
    b2lo = jnp.tile(b2[:16], 8)[None, :]
    b2hi = jnp.tile(b2[16:], 8)[None, :]

    degp = _sc_deg(dst2d, z1)
    x_flat = jnp.pad(x, ((0, N_ACC - N), (0, 0))).reshape(8 * RS, 128)
    s1, g0_flat = _tc1(degp.reshape(2 * RS, 128), x_flat, k8)

    q = _sc_w8(g0_flat.reshape(N_ACC, 8), src2d, dst2d, z8)
    glo_f, ghi_f = _tc2(q.reshape(16 * RS, 128), g0_flat, s1, k8, k16,
                        bd1, b1w, bd2lo, bd2hi)

    g2tab = jnp.concatenate([glo_f, ghi_f]).reshape(2 * N_ACC, 16)
    p16 = _sc_w16(g2tab, src2c, dst2d, z16)
    g3rep = _tc3(p16.reshape(32 * RS, 128), glo_f, ghi_f, s1, k16,
                 m16lo, m16hi, b2lo, b2hi)

    r = _sc_w1(g3rep.reshape(16 * N_ACC), srcw1, dstw1, z1)
    batchf = jnp.pad(batch, (0, N_ACC - N),
                     constant_values=G).reshape(RS, 128)
    return _tc4(r.reshape(2 * RS, 128), s1, b3[None, :], batchf)


# speedup vs baseline: 39.7537x; 1.0015x over previous
import functools

import jax
import jax.numpy as jnp
from jax import lax
from jax.experimental import pallas as pl
from jax.experimental.pallas import tpu as pltpu
from jax.experimental.pallas import tpu_sc as plsc

N = 100000
E = 1600000
G = 64
NC, NS = 2, 16
N_ACC = 100352
ROWS_PT = N_ACC // NS
KCH = 8
EP = 1638400
EPS = 1769472
BM = 2048
STEPS = N_ACC // BM
RS = N_ACC // 128


def _make_sc_agg(w, core_split, ep, kch=2, ch=4):
    tiles = NS if core_split else NC * NS
    ept = ep // tiles
    cs = ch * 128
    spt = ept // cs
    nblk = spt // kch
    assert nblk % 2 == 0 and spt % kch == 0 and ept % cs == 0
    acc_shape = (N_ACC,) if w is None else (N_ACC, w)
    rows_shape = (2, kch * cs) if w is None else (2, kch * cs, w)
    out_rows = NC * N_ACC
    out_shape = (out_rows,) if w is None else (out_rows, w)
    mesh = plsc.VectorSubcoreMesh(core_axis_name="c", subcore_axis_name="s")

    @functools.partial(
        pl.kernel,
        mesh=mesh,
        out_type=jax.ShapeDtypeStruct(out_shape, jnp.float32),
        compiler_params=pltpu.CompilerParams(use_tc_tiling_on_sc=False),
        scratch_types=[
            pltpu.VMEM((2, kch, cs), jnp.int32),
            pltpu.VMEM((2, kch, cs), jnp.int32),
            pltpu.VMEM(rows_shape, jnp.float32),
            pltpu.VMEM_SHARED(acc_shape, jnp.float32),
            pltpu.SemaphoreType.DMA,
            pltpu.SemaphoreType.DMA,
            pltpu.SemaphoreType.DMA,
            pltpu.SemaphoreType.DMA,
        ],
    )
    def kern(tab, src2d, dst2d, zeros, out, src_i, dst_i, rows, acc,
             sg0, sg1, ss0, ss1):
        c = lax.axis_index("c")
        s = lax.axis_index("s")
        pltpu.sync_copy(zeros.at[pl.ds(s * ROWS_PT, ROWS_PT)],
                        acc.at[pl.ds(s * ROWS_PT, ROWS_PT)])
        plsc.subcore_barrier()
        tile = s if core_split else s * NC + c
        dst_ch0 = tile * spt
        src_ch0 = c * (ep // cs) + dst_ch0 if core_split else dst_ch0
        sg = (sg0, sg1)
        ss = (ss0, ss1)

        def row_slot(b, j):
            if w is None:
                return rows.at[b, pl.ds(j * cs, cs)]
            return rows.at[b, pl.ds(j * cs, cs), :]

        def idx_slot(ref, b, j):
            return ref.at[b, j]

        def load_idx(k, b):
            pltpu.sync_copy(src2d.at[pl.ds(src_ch0 + k * kch, kch)],
                            src_i.at[b])
            pltpu.sync_copy(dst2d.at[pl.ds(dst_ch0 + k * kch, kch)],
                            dst_i.at[b])

        def start_gathers(b):
            for j in range(kch):
                pltpu.async_copy(tab.at[idx_slot(src_i, b, j)],
                                 row_slot(b, j), sg[b])

        def drain_gathers(b):
            for j in range(kch):
                pltpu.make_async_copy(tab.at[idx_slot(src_i, b, j)],
                                      row_slot(b, j), sg[b]).wait()

        def start_scatters(b):
            for j in range(kch):
                pltpu.async_copy(row_slot(b, j),
                                 acc.at[idx_slot(dst_i, b, j)],
                                 ss[b], add=True)

        def drain_scatters(b):
            for j in range(kch):
                pltpu.make_async_copy(row_slot(b, j),
                                      acc.at[idx_slot(dst_i, b, j)],
                                      ss[b]).wait()

        load_idx(0, 0)
        start_gathers(0)

        def phase(k, b):
            nb = 1 - b

            @pl.when(k >= 1)
            def _():
                drain_scatters(nb)

            @pl.when(k + 1 < nblk)
            def _():
                load_idx(k + 1, nb)
                start_gathers(nb)

            drain_gathers(b)
            start_scatters(b)

        def body(uu, carry):
            phase(2 * uu, 0)
            phase(2 * uu + 1, 1)
            return carry

        lax.fori_loop(0, nblk // 2, body, 0)
        drain_scatters(1)
        plsc.subcore_barrier()
        pltpu.sync_copy(acc.at[pl.ds(s * ROWS_PT, ROWS_PT)],
                        out.at[pl.ds(c * N_ACC + s * ROWS_PT, ROWS_PT)])

    return kern


def _make_sc_deg():
    kch = 2
    cs = 512
    ept = EP // (NC * NS)
    spt = ept // cs
    nblk = spt // kch
    assert nblk % 2 == 0
    mesh = plsc.VectorSubcoreMesh(core_axis_name="c", subcore_axis_name="s")

    @functools.partial(
        pl.kernel,
        mesh=mesh,
        out_type=jax.ShapeDtypeStruct((NC * N_ACC,), jnp.float32),
        compiler_params=pltpu.CompilerParams(use_tc_tiling_on_sc=False),
        scratch_types=[
            pltpu.VMEM((2, kch, cs), jnp.int32),
            pltpu.VMEM((cs,), jnp.float32),
            pltpu.VMEM_SHARED((N_ACC,), jnp.float32),
            pltpu.SemaphoreType.DMA,
            pltpu.SemaphoreType.DMA,
        ],
    )
    def kern(dst2d, zeros, out, dst_i, ones_v, acc, ss0, ss1):
        c = lax.axis_index("c")
        s = lax.axis_index("s")
        for k in range(cs // 16):
            ones_v[pl.ds(16 * k, 16)] = jnp.ones((16,), jnp.float32)
        pltpu.sync_copy(zeros.at[pl.ds(s * ROWS_PT, ROWS_PT)],
                        acc.at[pl.ds(s * ROWS_PT, ROWS_PT)])
        plsc.subcore_barrier()
        tile = s * NC + c
        ch0 = tile * spt
        ss = (ss0, ss1)

        def idx_slot(b, j):
            return dst_i.at[b, j]

        def start_scatters(b):
            for j in range(kch):
                pltpu.async_copy(ones_v, acc.at[idx_slot(b, j)], ss[b],
                                 add=True)

        def drain_scatters(b):
            for j in range(kch):
                pltpu.make_async_copy(ones_v, acc.at[idx_slot(b, j)],
                                      ss[b]).wait()

        pltpu.sync_copy(dst2d.at[pl.ds(ch0, kch)], dst_i.at[0])

        def phase(k, b):
            nb = 1 - b

            @pl.when(k >= 1)
            def _():
                drain_scatters(nb)

            @pl.when(k + 1 < nblk)
            def _():
                pltpu.sync_copy(
                    dst2d.at[pl.ds(ch0 + (k + 1) * kch, kch)],
                    dst_i.at[nb])

            start_scatters(b)

        def body(uu, carry):
            phase(2 * uu, 0)
            phase(2 * uu + 1, 1)
            return carry

        lax.fori_loop(0, nblk // 2, body, 0)
        drain_scatters(1)
        plsc.subcore_barrier()
        pltpu.sync_copy(acc.at[pl.ds(s * ROWS_PT, ROWS_PT)],
                        out.at[pl.ds(c * N_ACC + s * ROWS_PT, ROWS_PT)])

    return kern


_sc_deg = _make_sc_deg()
_sc_w8 = _make_sc_agg(8, core_split=False, ep=EP)
_sc_w16 = _make_sc_agg(16, core_split=True, ep=EP, kch=1)
_sc_w1 = _make_sc_agg(None, core_split=False, ep=EPS)


def _fs(rows, off=0):
    return pl.BlockSpec((rows, 128), lambda i, o=off: (i + o, 0))


def _full(shape):
    return pl.BlockSpec(shape, lambda i: tuple(0 for _ in shape))


def _tc1_body(d0, d1, x, k8, s_o, g0_o):
    s = lax.rsqrt(d0[...] + d1[...] + 1.0)
    s_o[...] = s
    v = jnp.reshape(s, (128, 16))
    srep8 = lax.dot_general(v, k8[...], (((1,), (0,)), ((), ())),
                            preferred_element_type=jnp.float32)
    g0_o[...] = x[...] * srep8


def _tc1(degf, x_flat, k8):
    return pl.pallas_call(
        _tc1_body,
        grid=(STEPS,),
        in_specs=[_fs(16), _fs(16, STEPS), _fs(128), _full((16, 128))],
        out_specs=[_fs(16), _fs(128)],
        out_shape=[
            jax.ShapeDtypeStruct((RS, 128), jnp.float32),
            jax.ShapeDtypeStruct((8 * RS, 128), jnp.float32),
        ],
    )(degf, degf, x_flat, k8)


def _tc2_body(q0, q1, g0, s, k8, k16, bd1, b1w, bd2lo, bd2hi, lo_o, hi_o):
    v = jnp.reshape(s[...], (128, 16))
    srep8 = lax.dot_general(v, k8[...], (((1,), (0,)), ((), ())),
                            preferred_element_type=jnp.float32)
    a1f = srep8 * (q0[...] + q1[...] + g0[...])
    h1w = lax.dot_general(a1f, bd1[...], (((1,), (0,)), ((), ())),
                          preferred_element_type=jnp.float32)
    h1w = jnp.maximum(h1w + b1w[...], 0.0)
    srepw16 = lax.dot_general(v, k16[...], (((1,), (0,)), ((), ())),
                              preferred_element_type=jnp.float32)
    p2lo = lax.dot_general(h1w, bd2lo[...], (((1,), (0,)), ((), ())),
                           preferred_element_type=jnp.float32)
    p2hi = lax.dot_general(h1w, bd2hi[...], (((1,), (0,)), ((), ())),
                           preferred_element_type=jnp.float32)
    lo_o[...] = jnp.reshape(srepw16 * p2lo, (256, 128))
    hi_o[...] = jnp.reshape(srepw16 * p2hi, (256, 128))


def _tc2(qf, g0_flat, s1, k8, k16, bd1, b1w, bd2lo, bd2hi):
    return pl.pallas_call(
        _tc2_body,
        grid=(STEPS,),
        in_specs=[_fs(128), _fs(128, STEPS), _fs(128), _fs(16),
                  _full((16, 128)), _full((16, 256)), _full((128, 2048)),
                  _full((1, 2048)), _full((2048, 256)), _full((2048, 256))],
        out_specs=[_fs(256), _fs(256)],
        out_shape=[
            jax.ShapeDtypeStruct((16 * RS, 128), jnp.float32),
            jax.ShapeDtypeStruct((16 * RS, 128), jnp.float32),
        ],
    )(qf, qf, g0_flat, s1, k8, k16, bd1, b1w, bd2lo, bd2hi)


def _tc3_body(pc0, pc1, glo, ghi, s, k16, m16lo, m16hi, b2lo, b2hi, g3_o):
    v = jnp.reshape(s[...], (128, 16))
    srepw16 = lax.dot_general(v, k16[...], (((1,), (0,)), ((), ())),
                              preferred_element_type=jnp.float32)
    srep16 = jnp.reshape(srepw16, (256, 128))
    h2lo = jnp.maximum(srep16 * (pc0[...] + glo[...]) + b2lo[...], 0.0)
    h2hi = jnp.maximum(srep16 * (pc1[...] + ghi[...]) + b2hi[...], 0.0)
    p3rep = (lax.dot_general(h2lo, m16lo[...], (((1,), (0,)), ((), ())),
                             preferred_element_type=jnp.float32)
             + lax.dot_general(h2hi, m16hi[...], (((1,), (0,)), ((), ())),
                               preferred_element_type=jnp.float32))
    g3_o[...] = srep16 * p3rep


def _tc3(p16f, glo_f, ghi_f, s1, k16, m16lo, m16hi, b2lo, b2hi):
    return pl.pallas_call(
        _tc3_body,
        grid=(STEPS,),
        in_specs=[_fs(256), _fs(256, STEPS), _fs(256), _fs(256), _fs(16),
                  _full((16, 256)), _full((128, 128)), _full((128, 128)),
                  _full((1, 128)), _full((1, 128))],
        out_specs=_fs(256),
        out_shape=jax.ShapeDtypeStruct((16 * RS, 128), jnp.float32),
    )(p16f, p16f, glo_f, ghi_f, s1, k16, m16lo, m16hi, b2lo, b2hi)


def _tc4_body(r0, r1, s, b3, bat, out, acc):
    i = pl.program_id(0)
    h3 = s[...] * (r0[...] + r1[...]) + b3[...]
    h3row = jnp.reshape(h3, (1, BM))
    brow = jnp.reshape(bat[...], (1, BM))
    gid = lax.broadcasted_iota(jnp.int32, (G, BM), 0)
    mask = (brow == gid).astype(jnp.float32)
    part_s = jnp.sum(mask * h3row, axis=1, keepdims=True)
    part_c = jnp.sum(mask, axis=1, keepdims=True)
    part = jnp.concatenate([part_s, part_c], axis=1)

    @pl.when(i == 0)
    def _():
        acc[...] = jnp.zeros_like(acc)

    acc[...] += part

    @pl.when(i == STEPS - 1)
    def _():
        sums = acc[:, 0:1]
        cnts = jnp.maximum(acc[:, 1:2], 1.0)
        out[...] = jax.nn.sigmoid(sums / cnts)


def _tc4(rf, s1, b3r, batchf):
    return pl.pallas_call(
        _tc4_body,
        grid=(STEPS,),
        in_specs=[_fs(16), _fs(16, STEPS), _fs(16), _full((1, 1)), _fs(16)],
        out_specs=_full((G, 1)),
        out_shape=jax.ShapeDtypeStruct((G, 1), jnp.float32),
        scratch_shapes=[pltpu.VMEM((G, 2), jnp.float32)],
    )(rf, rf, s1, b3r, batchf)


def kernel(x, edge_index, batch, W1, b1, W2, b2, W3, b3):
    src = edge_index[0]
    dst = edge_index[1]
    i16 = jnp.eye(16, dtype=jnp.float32)
    src_p = jnp.concatenate([src, jnp.zeros((EP - E,), jnp.int32)])
    dst_p = jnp.concatenate([dst, jnp.full((EP - E,), N, jnp.int32)])
    src2d = src_p.reshape(-1, 512)
    dst2d = dst_p.reshape(-1, 512)
    src2c = jnp.concatenate([src_p, src_p + N_ACC]).reshape(-1, 512)
    loop = jnp.arange(N_ACC, dtype=jnp.int32)
    srcw1 = jnp.concatenate(
        [src * 16, loop * 16,
         jnp.zeros((EPS - E - N_ACC,), jnp.int32)]).reshape(-1, 512)
    dstw1 = jnp.concatenate(
        [dst, loop, jnp.full((EPS - E - N_ACC,), N, jnp.int32)]
    ).reshape(-1, 512)
    z1 = jnp.zeros((N_ACC,), jnp.float32)
    z8 = jnp.zeros((N_ACC, 8), jnp.float32)
    z16 = jnp.zeros((N_ACC, 16), jnp.float32)
    k8 = jnp.repeat(i16, 8, axis=1)
    k16 = jnp.repeat(i16, 16, axis=1)
    bd1 = jnp.kron(i16, W1.T)
    b1w = jnp.tile(b1, 16)[None, :]
    bd2lo = jnp.kron(i16, W2.T[:, :16])
    bd2hi = jnp.kron(i16, W2.T[:, 16:])
    i8 = jnp.eye(8, dtype=jnp.float32)
    m16lo = jnp.kron(i8, jnp.outer(W3[0, :16], jnp.ones(16, jnp.float32)))
    m16hi = jnp.kron(i8, jnp.outer(W3[0, 16:], jnp.ones(16, jnp.float32)))
    b2lo = jnp.tile(b2[:16], 8)[None, :]
    b2hi = jnp.tile(b2[16:], 8)[None, :]

    degp = _sc_deg(dst2d, z1)
    x_flat = jnp.pad(x, ((0, N_ACC - N), (0, 0))).reshape(8 * RS, 128)
    s1, g0_flat = _tc1(degp.reshape(2 * RS, 128), x_flat, k8)

    q = _sc_w8(g0_flat.reshape(N_ACC, 8), src2d, dst2d, z8)
    glo_f, ghi_f = _tc2(q.reshape(16 * RS, 128), g0_flat, s1, k8, k16,
                        bd1, b1w, bd2lo, bd2hi)

    g2tab = jnp.concatenate([glo_f, ghi_f]).reshape(2 * N_ACC, 16)
    p16 = _sc_w16(g2tab, src2c, dst2d, z16)
    g3rep = _tc3(p16.reshape(32 * RS, 128), glo_f, ghi_f, s1, k16,
                 m16lo, m16hi, b2lo, b2hi)

    r = _sc_w1(g3rep.reshape(16 * N_ACC), srcw1, dstw1, z1)
    batchf = jnp.pad(batch, (0, N_ACC - N),
                     constant_values=G).reshape(RS, 128)
    return _tc4(r.reshape(2 * RS, 128), s1, b3[None, :], batchf)


# speedup vs baseline: 40.4424x; 1.0173x over previous
import functools

import jax
import jax.numpy as jnp
from jax import lax
from jax.experimental import pallas as pl
from jax.experimental.pallas import tpu as pltpu
from jax.experimental.pallas import tpu_sc as plsc

N = 100000
E = 1600000
G = 64
NC, NS = 2, 16
N_ACC = 100352
ROWS_PT = N_ACC // NS
KCH = 8
EP = 1638400
EPS = 1769472
BM = 2048
STEPS = N_ACC // BM
RS = N_ACC // 128


def _make_sc_agg(w, core_split, ep, kch=2, ch=4):
    tiles = NS if core_split else NC * NS
    ept = ep // tiles
    cs = ch * 128
    spt = ept // cs
    nblk = spt // kch
    assert nblk % 2 == 0 and spt % kch == 0 and ept % cs == 0
    acc_shape = (N_ACC,) if w is None else (N_ACC, w)
    rows_shape = (2, kch * cs) if w is None else (2, kch * cs, w)
    out_rows = NC * N_ACC
    out_shape = (out_rows,) if w is None else (out_rows, w)
    mesh = plsc.VectorSubcoreMesh(core_axis_name="c", subcore_axis_name="s")

    @functools.partial(
        pl.kernel,
        mesh=mesh,
        out_type=jax.ShapeDtypeStruct(out_shape, jnp.float32),
        compiler_params=pltpu.CompilerParams(use_tc_tiling_on_sc=False),
        scratch_types=[
            pltpu.VMEM((2, kch, cs), jnp.int32),
            pltpu.VMEM((2, kch, cs), jnp.int32),
            pltpu.VMEM(rows_shape, jnp.float32),
            pltpu.VMEM_SHARED(acc_shape, jnp.float32),
            pltpu.SemaphoreType.DMA,
            pltpu.SemaphoreType.DMA,
            pltpu.SemaphoreType.DMA,
            pltpu.SemaphoreType.DMA,
        ],
    )
    def kern(tab, src2d, dst2d, zeros, out, src_i, dst_i, rows, acc,
             sg0, sg1, ss0, ss1):
        c = lax.axis_index("c")
        s = lax.axis_index("s")
        pltpu.sync_copy(zeros.at[pl.ds(s * ROWS_PT, ROWS_PT)],
                        acc.at[pl.ds(s * ROWS_PT, ROWS_PT)])
        plsc.subcore_barrier()
        tile = s if core_split else s * NC + c
        dst_ch0 = tile * spt
        src_ch0 = c * (ep // cs) + dst_ch0 if core_split else dst_ch0
        sg = (sg0, sg1)
        ss = (ss0, ss1)

        def row_slot(b, j):
            if w is None:
                return rows.at[b, pl.ds(j * cs, cs)]
            return rows.at[b, pl.ds(j * cs, cs), :]

        def idx_slot(ref, b, j):
            return ref.at[b, j]

        def load_idx(k, b):
            pltpu.sync_copy(src2d.at[pl.ds(src_ch0 + k * kch, kch)],
                            src_i.at[b])
            pltpu.sync_copy(dst2d.at[pl.ds(dst_ch0 + k * kch, kch)],
                            dst_i.at[b])

        def start_gathers(b):
            for j in range(kch):
                pltpu.async_copy(tab.at[idx_slot(src_i, b, j)],
                                 row_slot(b, j), sg[b])

        def drain_gathers(b):
            for j in range(kch):
                pltpu.make_async_copy(tab.at[idx_slot(src_i, b, j)],
                                      row_slot(b, j), sg[b]).wait()

        def start_scatters(b):
            for j in range(kch):
                pltpu.async_copy(row_slot(b, j),
                                 acc.at[idx_slot(dst_i, b, j)],
                                 ss[b], add=True)

        def drain_scatters(b):
            for j in range(kch):
                pltpu.make_async_copy(row_slot(b, j),
                                      acc.at[idx_slot(dst_i, b, j)],
                                      ss[b]).wait()

        load_idx(0, 0)
        start_gathers(0)

        def phase(k, b):
            nb = 1 - b

            @pl.when(k >= 1)
            def _():
                drain_scatters(nb)

            @pl.when(k + 1 < nblk)
            def _():
                load_idx(k + 1, nb)
                start_gathers(nb)

            drain_gathers(b)
            start_scatters(b)

        def body(uu, carry):
            phase(2 * uu, 0)
            phase(2 * uu + 1, 1)
            return carry

        lax.fori_loop(0, nblk // 2, body, 0)
        drain_scatters(1)
        plsc.subcore_barrier()
        pltpu.sync_copy(acc.at[pl.ds(s * ROWS_PT, ROWS_PT)],
                        out.at[pl.ds(c * N_ACC + s * ROWS_PT, ROWS_PT)])

    return kern


def _make_sc_deg():
    kch = 2
    cs = 512
    ept = EP // (NC * NS)
    spt = ept // cs
    nblk = spt // kch
    assert nblk % 2 == 0
    mesh = plsc.VectorSubcoreMesh(core_axis_name="c", subcore_axis_name="s")

    @functools.partial(
        pl.kernel,
        mesh=mesh,
        out_type=jax.ShapeDtypeStruct((NC * N_ACC,), jnp.float32),
        compiler_params=pltpu.CompilerParams(use_tc_tiling_on_sc=False),
        scratch_types=[
            pltpu.VMEM((2, kch, cs), jnp.int32),
            pltpu.VMEM((cs,), jnp.float32),
            pltpu.VMEM_SHARED((N_ACC,), jnp.float32),
            pltpu.SemaphoreType.DMA,
            pltpu.SemaphoreType.DMA,
        ],
    )
    def kern(dst2d, zeros, out, dst_i, ones_v, acc, ss0, ss1):
        c = lax.axis_index("c")
        s = lax.axis_index("s")
        for k in range(cs // 16):
            ones_v[pl.ds(16 * k, 16)] = jnp.ones((16,), jnp.float32)
        pltpu.sync_copy(zeros.at[pl.ds(s * ROWS_PT, ROWS_PT)],
                        acc.at[pl.ds(s * ROWS_PT, ROWS_PT)])
        plsc.subcore_barrier()
        tile = s * NC + c
        ch0 = tile * spt
        ss = (ss0, ss1)

        def idx_slot(b, j):
            return dst_i.at[b, j]

        def start_scatters(b):
            for j in range(kch):
                pltpu.async_copy(ones_v, acc.at[idx_slot(b, j)], ss[b],
                                 add=True)

        def drain_scatters(b):
            for j in range(kch):
                pltpu.make_async_copy(ones_v, acc.at[idx_slot(b, j)],
                                      ss[b]).wait()

        pltpu.sync_copy(dst2d.at[pl.ds(ch0, kch)], dst_i.at[0])

        def phase(k, b):
            nb = 1 - b

            @pl.when(k >= 1)
            def _():
                drain_scatters(nb)

            @pl.when(k + 1 < nblk)
            def _():
                pltpu.sync_copy(
                    dst2d.at[pl.ds(ch0 + (k + 1) * kch, kch)],
                    dst_i.at[nb])

            start_scatters(b)

        def body(uu, carry):
            phase(2 * uu, 0)
            phase(2 * uu + 1, 1)
            return carry

        lax.fori_loop(0, nblk // 2, body, 0)
        drain_scatters(1)
        plsc.subcore_barrier()
        pltpu.sync_copy(acc.at[pl.ds(s * ROWS_PT, ROWS_PT)],
                        out.at[pl.ds(c * N_ACC + s * ROWS_PT, ROWS_PT)])

    return kern


_sc_deg = _make_sc_deg()
_sc_w8 = _make_sc_agg(8, core_split=False, ep=EP)
_sc_w16 = _make_sc_agg(16, core_split=True, ep=EP, kch=1)
_sc_w1 = _make_sc_agg(None, core_split=False, ep=EPS)


def _fs(rows, off=0):
    return pl.BlockSpec((rows, 128), lambda i, o=off: (i + o, 0))


def _full(shape):
    return pl.BlockSpec(shape, lambda i: tuple(0 for _ in shape))


def _tc1_body(d0, d1, x, k8, s_o, g0_o):
    s = lax.rsqrt(d0[...] + d1[...] + 1.0)
    s_o[...] = s
    v = jnp.reshape(s, (128, 16))
    srep8 = lax.dot_general(v, k8[...], (((1,), (0,)), ((), ())),
                            preferred_element_type=jnp.float32)
    g0_o[...] = x[...] * srep8


def _tc1(degf, x_flat, k8):
    return pl.pallas_call(
        _tc1_body,
        grid=(STEPS,),
        in_specs=[_fs(16), _fs(16, STEPS), _fs(128), _full((16, 128))],
        out_specs=[_fs(16), _fs(128)],
        out_shape=[
            jax.ShapeDtypeStruct((RS, 128), jnp.float32),
            jax.ShapeDtypeStruct((8 * RS, 128), jnp.float32),
        ],
    )(degf, degf, x_flat, k8)


def _tc2_body(q0, q1, g0, s, k8, k16, bd1, b1w, bd2lo, bd2hi, lo_o, hi_o):
    v = jnp.reshape(s[...], (128, 16))
    srep8 = lax.dot_general(v, k8[...], (((1,), (0,)), ((), ())),
                            preferred_element_type=jnp.float32)
    a1f = srep8 * (q0[...] + q1[...] + g0[...])
    h1w = lax.dot_general(a1f, bd1[...], (((1,), (0,)), ((), ())),
                          preferred_element_type=jnp.float32)
    h1w = jnp.maximum(h1w + b1w[...], 0.0)
    srepw16 = lax.dot_general(v, k16[...], (((1,), (0,)), ((), ())),
                              preferred_element_type=jnp.float32)
    p2lo = lax.dot_general(h1w, bd2lo[...], (((1,), (0,)), ((), ())),
                           preferred_element_type=jnp.float32)
    p2hi = lax.dot_general(h1w, bd2hi[...], (((1,), (0,)), ((), ())),
                           preferred_element_type=jnp.float32)
    lo_o[...] = jnp.reshape(srepw16 * p2lo, (256, 128))
    hi_o[...] = jnp.reshape(srepw16 * p2hi, (256, 128))


def _tc2(qf, g0_flat, s1, k8, k16, bd1, b1w, bd2lo, bd2hi):
    return pl.pallas_call(
        _tc2_body,
        grid=(STEPS,),
        in_specs=[_fs(128), _fs(128, STEPS), _fs(128), _fs(16),
                  _full((16, 128)), _full((16, 256)), _full((128, 2048)),
                  _full((1, 2048)), _full((2048, 256)), _full((2048, 256))],
        out_specs=[_fs(256), _fs(256)],
        out_shape=[
            jax.ShapeDtypeStruct((16 * RS, 128), jnp.float32),
            jax.ShapeDtypeStruct((16 * RS, 128), jnp.float32),
        ],
    )(qf, qf, g0_flat, s1, k8, k16, bd1, b1w, bd2lo, bd2hi)


def _tc3_body(pc0, pc1, glo, ghi, s, k16, m16lo, m16hi, b2lo, b2hi, g3_o):
    v = jnp.reshape(s[...], (128, 16))
    srepw16 = lax.dot_general(v, k16[...], (((1,), (0,)), ((), ())),
                              preferred_element_type=jnp.float32)
    srep16 = jnp.reshape(srepw16, (256, 128))
    h2lo = jnp.maximum(srep16 * (pc0[...] + glo[...]) + b2lo[...], 0.0)
    h2hi = jnp.maximum(srep16 * (pc1[...] + ghi[...]) + b2hi[...], 0.0)
    p3rep = (lax.dot_general(h2lo, m16lo[...], (((1,), (0,)), ((), ())),
                             preferred_element_type=jnp.float32)
             + lax.dot_general(h2hi, m16hi[...], (((1,), (0,)), ((), ())),
                               preferred_element_type=jnp.float32))
    g3_o[...] = srep16 * p3rep


def _tc3(p16f, glo_f, ghi_f, s1, k16, m16lo, m16hi, b2lo, b2hi):
    return pl.pallas_call(
        _tc3_body,
        grid=(STEPS,),
        in_specs=[_fs(256), _fs(256, STEPS), _fs(256), _fs(256), _fs(16),
                  _full((16, 256)), _full((128, 128)), _full((128, 128)),
                  _full((1, 128)), _full((1, 128))],
        out_specs=_fs(256),
        out_shape=jax.ShapeDtypeStruct((16 * RS, 128), jnp.float32),
    )(p16f, p16f, glo_f, ghi_f, s1, k16, m16lo, m16hi, b2lo, b2hi)


def _tc4_body(r0, r1, s, b3, bat, out, acc):
    i = pl.program_id(0)
    h3 = s[...] * (r0[...] + r1[...]) + b3[...]
    h3row = jnp.reshape(h3, (1, BM))
    brow = jnp.reshape(bat[...], (1, BM))
    gid = lax.broadcasted_iota(jnp.int32, (G, BM), 0)
    mask = (brow == gid).astype(jnp.float32)
    part_s = jnp.sum(mask * h3row, axis=1, keepdims=True)
    part_c = jnp.sum(mask, axis=1, keepdims=True)
    part = jnp.concatenate([part_s, part_c], axis=1)

    @pl.when(i == 0)
    def _():
        acc[...] = jnp.zeros_like(acc)

    acc[...] += part

    @pl.when(i == STEPS - 1)
    def _():
        sums = acc[:, 0:1]
        cnts = jnp.maximum(acc[:, 1:2], 1.0)
        out[...] = jax.nn.sigmoid(sums / cnts)


def _tc4(rf, s1, b3r, batchf):
    return pl.pallas_call(
        _tc4_body,
        grid=(STEPS,),
        in_specs=[_fs(16), _fs(16, STEPS), _fs(16), _full((1, 1)), _fs(16)],
        out_specs=_full((G, 1)),
        out_shape=jax.ShapeDtypeStruct((G, 1), jnp.float32),
        scratch_shapes=[pltpu.VMEM((G, 2), jnp.float32)],
    )(rf, rf, s1, b3r, batchf)


def kernel(x, edge_index, batch, W1, b1, W2, b2, W3, b3):
    src = edge_index[0]
    dst = edge_index[1]
    i16 = jnp.eye(16, dtype=jnp.float32)
    spare = N_ACC - N
    trash = N + jnp.arange(EP - E, dtype=jnp.int32) % spare
    src_p = jnp.concatenate([src, jnp.zeros((EP - E,), jnp.int32)])
    dst_p = jnp.concatenate([dst, trash])
    src2d = src_p.reshape(-1, 512)
    dst2d = dst_p.reshape(-1, 512)
    src2c = jnp.concatenate([src_p, src_p + N_ACC]).reshape(-1, 512)
    loop = jnp.arange(N_ACC, dtype=jnp.int32)
    srcw1 = jnp.concatenate(
        [src * 16, loop * 16,
         jnp.zeros((EPS - E - N_ACC,), jnp.int32)]).reshape(-1, 512)
    trashw1 = N + jnp.arange(EPS - E - N_ACC, dtype=jnp.int32) % spare
    dstw1 = jnp.concatenate([dst, loop, trashw1]).reshape(-1, 512)
    z1 = jnp.zeros((N_ACC,), jnp.float32)
    z8 = jnp.zeros((N_ACC, 8), jnp.float32)
    z16 = jnp.zeros((N_ACC, 16), jnp.float32)
    k8 = jnp.repeat(i16, 8, axis=1)
    k16 = jnp.repeat(i16, 16, axis=1)
    bd1 = jnp.kron(i16, W1.T)
    b1w = jnp.tile(b1, 16)[None, :]
    bd2lo = jnp.kron(i16, W2.T[:, :16])
    bd2hi = jnp.kron(i16, W2.T[:, 16:])
    i8 = jnp.eye(8, dtype=jnp.float32)
    m16lo = jnp.kron(i8, jnp.outer(W3[0, :16], jnp.ones(16, jnp.float32)))
    m16hi = jnp.kron(i8, jnp.outer(W3[0, 16:], jnp.ones(16, jnp.float32)))
    b2lo = jnp.tile(b2[:16], 8)[None, :]
    b2hi = jnp.tile(b2[16:], 8)[None, :]

    degp = _sc_deg(dst2d, z1)
    x_flat = jnp.pad(x, ((0, N_ACC - N), (0, 0))).reshape(8 * RS, 128)
    s1, g0_flat = _tc1(degp.reshape(2 * RS, 128), x_flat, k8)

    q = _sc_w8(g0_flat.reshape(N_ACC, 8), src2d, dst2d, z8)
    glo_f, ghi_f = _tc2(q.reshape(16 * RS, 128), g0_flat, s1, k8, k16,
                        bd1, b1w, bd2lo, bd2hi)

    g2tab = jnp.concatenate([glo_f, ghi_f]).reshape(2 * N_ACC, 16)
    p16 = _sc_w16(g2tab, src2c, dst2d, z16)
    g3rep = _tc3(p16.reshape(32 * RS, 128), glo_f, ghi_f, s1, k16,
                 m16lo, m16hi, b2lo, b2hi)

    r = _sc_w1(g3rep.reshape(16 * N_ACC), srcw1, dstw1, z1)
    batchf = jnp.pad(batch, (0, N_ACC - N),
                     constant_values=G).reshape(RS, 128)
    return _tc4(r.reshape(2 * RS, 128), s1, b3[None, :], batchf)


# speedup vs baseline: 65.9507x; 1.6307x over previous
import functools

import jax
import jax.numpy as jnp
from jax import lax
from jax.experimental import pallas as pl
from jax.experimental.pallas import tpu as pltpu
from jax.experimental.pallas import tpu_sc as plsc

N = 100000
E = 1600000
G = 64
NC, NS = 2, 16
N_ACC = 100352
ROWS_PT = N_ACC // NS
KCH = 8
EP = 1638400
EPS = 1769472
BM = 2048
STEPS = N_ACC // BM
RS = N_ACC // 128


def _make_sc_agg(w, core_split, ep, kch=2, ch=4):
    tiles = NS if core_split else NC * NS
    ept = ep // tiles
    cs = ch * 128
    spt = ept // cs
    nblk = spt // kch
    assert nblk % 2 == 0 and spt % kch == 0 and ept % cs == 0
    acc_shape = (N_ACC,) if w is None else (N_ACC, w)
    rows_shape = (2, kch * cs) if w is None else (2, kch * cs, w)
    out_rows = NC * N_ACC
    out_shape = (out_rows,) if w is None else (out_rows, w)
    mesh = plsc.VectorSubcoreMesh(core_axis_name="c", subcore_axis_name="s")

    @functools.partial(
        pl.kernel,
        mesh=mesh,
        out_type=jax.ShapeDtypeStruct(out_shape, jnp.float32),
        compiler_params=pltpu.CompilerParams(use_tc_tiling_on_sc=False),
        scratch_types=[
            pltpu.VMEM((2, kch, cs), jnp.int32),
            pltpu.VMEM((2, kch, cs), jnp.int32),
            pltpu.VMEM(rows_shape, jnp.float32),
            pltpu.VMEM_SHARED(acc_shape, jnp.float32),
            pltpu.SemaphoreType.DMA,
            pltpu.SemaphoreType.DMA,
            pltpu.SemaphoreType.DMA,
            pltpu.SemaphoreType.DMA,
        ],
    )
    def kern(tab, src2d, dst2d, zeros, out, src_i, dst_i, rows, acc,
             sg0, sg1, ss0, ss1):
        c = lax.axis_index("c")
        s = lax.axis_index("s")
        pltpu.sync_copy(zeros.at[pl.ds(s * ROWS_PT, ROWS_PT)],
                        acc.at[pl.ds(s * ROWS_PT, ROWS_PT)])
        plsc.subcore_barrier()
        tile = s if core_split else s * NC + c
        dst_ch0 = tile * spt
        src_ch0 = c * (ep // cs) + dst_ch0 if core_split else dst_ch0
        sg = (sg0, sg1)
        ss = (ss0, ss1)

        def row_slot(b, j):
            if w is None:
                return rows.at[b, pl.ds(j * cs, cs)]
            return rows.at[b, pl.ds(j * cs, cs), :]

        def idx_slot(ref, b, j):
            return ref.at[b, j]

        def load_idx(k, b):
            pltpu.sync_copy(src2d.at[pl.ds(src_ch0 + k * kch, kch)],
                            src_i.at[b])
            pltpu.sync_copy(dst2d.at[pl.ds(dst_ch0 + k * kch, kch)],
                            dst_i.at[b])

        def start_gathers(b):
            for j in range(kch):
                pltpu.async_copy(tab.at[idx_slot(src_i, b, j)],
                                 row_slot(b, j), sg[b])

        def drain_gathers(b):
            for j in range(kch):
                pltpu.make_async_copy(tab.at[idx_slot(src_i, b, j)],
                                      row_slot(b, j), sg[b]).wait()

        def start_scatters(b):
            for j in range(kch):
                pltpu.async_copy(row_slot(b, j),
                                 acc.at[idx_slot(dst_i, b, j)],
                                 ss[b], add=True)

        def drain_scatters(b):
            for j in range(kch):
                pltpu.make_async_copy(row_slot(b, j),
                                      acc.at[idx_slot(dst_i, b, j)],
                                      ss[b]).wait()

        load_idx(0, 0)
        start_gathers(0)

        def phase(k, b):
            nb = 1 - b

            @pl.when(k >= 1)
            def _():
                drain_scatters(nb)

            @pl.when(k + 1 < nblk)
            def _():
                load_idx(k + 1, nb)
                start_gathers(nb)

            drain_gathers(b)
            start_scatters(b)

        def body(uu, carry):
            phase(2 * uu, 0)
            phase(2 * uu + 1, 1)
            return carry

        lax.fori_loop(0, nblk // 2, body, 0)
        drain_scatters(1)
        plsc.subcore_barrier()
        pltpu.sync_copy(acc.at[pl.ds(s * ROWS_PT, ROWS_PT)],
                        out.at[pl.ds(c * N_ACC + s * ROWS_PT, ROWS_PT)])

    return kern


def _make_sc_deg():
    kch = 2
    cs = 512
    ept = EP // (NC * NS)
    spt = ept // cs
    nblk = spt // kch
    assert nblk % 2 == 0
    mesh = plsc.VectorSubcoreMesh(core_axis_name="c", subcore_axis_name="s")

    @functools.partial(
        pl.kernel,
        mesh=mesh,
        out_type=jax.ShapeDtypeStruct((NC * N_ACC,), jnp.float32),
        compiler_params=pltpu.CompilerParams(use_tc_tiling_on_sc=False),
        scratch_types=[
            pltpu.VMEM((2, kch, cs), jnp.int32),
            pltpu.VMEM((cs,), jnp.float32),
            pltpu.VMEM_SHARED((N_ACC,), jnp.float32),
            pltpu.SemaphoreType.DMA,
            pltpu.SemaphoreType.DMA,
        ],
    )
    def kern(dst2d, zeros, out, dst_i, ones_v, acc, ss0, ss1):
        c = lax.axis_index("c")
        s = lax.axis_index("s")
        for k in range(cs // 16):
            ones_v[pl.ds(16 * k, 16)] = jnp.ones((16,), jnp.float32)
        pltpu.sync_copy(zeros.at[pl.ds(s * ROWS_PT, ROWS_PT)],
                        acc.at[pl.ds(s * ROWS_PT, ROWS_PT)])
        plsc.subcore_barrier()
        tile = s * NC + c
        ch0 = tile * spt
        ss = (ss0, ss1)

        def idx_slot(b, j):
            return dst_i.at[b, j]

        def start_scatters(b):
            for j in range(kch):
                pltpu.async_copy(ones_v, acc.at[idx_slot(b, j)], ss[b],
                                 add=True)

        def drain_scatters(b):
            for j in range(kch):
                pltpu.make_async_copy(ones_v, acc.at[idx_slot(b, j)],
                                      ss[b]).wait()

        pltpu.sync_copy(dst2d.at[pl.ds(ch0, kch)], dst_i.at[0])

        def phase(k, b):
            nb = 1 - b

            @pl.when(k >= 1)
            def _():
                drain_scatters(nb)

            @pl.when(k + 1 < nblk)
            def _():
                pltpu.sync_copy(
                    dst2d.at[pl.ds(ch0 + (k + 1) * kch, kch)],
                    dst_i.at[nb])

            start_scatters(b)

        def body(uu, carry):
            phase(2 * uu, 0)
            phase(2 * uu + 1, 1)
            return carry

        lax.fori_loop(0, nblk // 2, body, 0)
        drain_scatters(1)
        plsc.subcore_barrier()
        pltpu.sync_copy(acc.at[pl.ds(s * ROWS_PT, ROWS_PT)],
                        out.at[pl.ds(c * N_ACC + s * ROWS_PT, ROWS_PT)])

    return kern


_sc_deg = _make_sc_deg()
_sc_w8 = _make_sc_agg(8, core_split=False, ep=EP)
_sc_w16 = _make_sc_agg(16, core_split=True, ep=EP, kch=1)
_sc_w1 = _make_sc_agg(None, core_split=False, ep=EPS)


def _fs(rows, off=0):
    return pl.BlockSpec((rows, 128), lambda i, o=off: (i + o, 0))


def _full(shape):
    return pl.BlockSpec(shape, lambda i: tuple(0 for _ in shape))


def _tc1_body(d0, d1, x, k8, s_o, g0_o):
    s = lax.rsqrt(d0[...] + d1[...] + 1.0)
    s_o[...] = s
    v = jnp.reshape(s, (128, 16))
    srep8 = lax.dot_general(v, k8[...], (((1,), (0,)), ((), ())),
                            preferred_element_type=jnp.float32)
    g0_o[...] = x[...] * srep8


def _tc1(degf, x_flat, k8):
    return pl.pallas_call(
        _tc1_body,
        grid=(STEPS,),
        in_specs=[_fs(16), _fs(16, STEPS), _fs(128), _full((16, 128))],
        out_specs=[_fs(16), _fs(128)],
        out_shape=[
            jax.ShapeDtypeStruct((RS, 128), jnp.float32),
            jax.ShapeDtypeStruct((8 * RS, 128), jnp.float32),
        ],
    )(degf, degf, x_flat, k8)


def _tc2_body(q0, q1, g0, s, k8, k16, bd1, b1w, bd2lo, bd2hi, lo_o, hi_o):
    v = jnp.reshape(s[...], (128, 16))
    srep8 = lax.dot_general(v, k8[...], (((1,), (0,)), ((), ())),
                            preferred_element_type=jnp.float32)
    a1f = srep8 * (q0[...] + q1[...] + g0[...])
    h1w = lax.dot_general(a1f, bd1[...], (((1,), (0,)), ((), ())),
                          preferred_element_type=jnp.float32)
    h1w = jnp.maximum(h1w + b1w[...], 0.0)
    srepw16 = lax.dot_general(v, k16[...], (((1,), (0,)), ((), ())),
                              preferred_element_type=jnp.float32)
    p2lo = lax.dot_general(h1w, bd2lo[...], (((1,), (0,)), ((), ())),
                           preferred_element_type=jnp.float32)
    p2hi = lax.dot_general(h1w, bd2hi[...], (((1,), (0,)), ((), ())),
                           preferred_element_type=jnp.float32)
    lo_o[...] = jnp.reshape(srepw16 * p2lo, (256, 128))
    hi_o[...] = jnp.reshape(srepw16 * p2hi, (256, 128))


def _tc2(qf, g0_flat, s1, k8, k16, bd1, b1w, bd2lo, bd2hi):
    return pl.pallas_call(
        _tc2_body,
        grid=(STEPS,),
        in_specs=[_fs(128), _fs(128, STEPS), _fs(128), _fs(16),
                  _full((16, 128)), _full((16, 256)), _full((128, 2048)),
                  _full((1, 2048)), _full((2048, 256)), _full((2048, 256))],
        out_specs=[_fs(256), _fs(256)],
        out_shape=[
            jax.ShapeDtypeStruct((16 * RS, 128), jnp.float32),
            jax.ShapeDtypeStruct((16 * RS, 128), jnp.float32),
        ],
    )(qf, qf, g0_flat, s1, k8, k16, bd1, b1w, bd2lo, bd2hi)


def _tc3_body(pc0, pc1, glo, ghi, s, k16, m16lo, m16hi, b2lo, b2hi, g3_o):
    v = jnp.reshape(s[...], (128, 16))
    srepw16 = lax.dot_general(v, k16[...], (((1,), (0,)), ((), ())),
                              preferred_element_type=jnp.float32)
    srep16 = jnp.reshape(srepw16, (256, 128))
    h2lo = jnp.maximum(srep16 * (pc0[...] + glo[...]) + b2lo[...], 0.0)
    h2hi = jnp.maximum(srep16 * (pc1[...] + ghi[...]) + b2hi[...], 0.0)
    p3rep = (lax.dot_general(h2lo, m16lo[...], (((1,), (0,)), ((), ())),
                             preferred_element_type=jnp.float32)
             + lax.dot_general(h2hi, m16hi[...], (((1,), (0,)), ((), ())),
                               preferred_element_type=jnp.float32))
    g3_o[...] = srep16 * p3rep


def _tc3(p16f, glo_f, ghi_f, s1, k16, m16lo, m16hi, b2lo, b2hi):
    return pl.pallas_call(
        _tc3_body,
        grid=(STEPS,),
        in_specs=[_fs(256), _fs(256, STEPS), _fs(256), _fs(256), _fs(16),
                  _full((16, 256)), _full((128, 128)), _full((128, 128)),
                  _full((1, 128)), _full((1, 128))],
        out_specs=_fs(256),
        out_shape=jax.ShapeDtypeStruct((16 * RS, 128), jnp.float32),
    )(p16f, p16f, glo_f, ghi_f, s1, k16, m16lo, m16hi, b2lo, b2hi)


def _tc4_body(r0, r1, s, b3, bat, out, acc):
    i = pl.program_id(0)
    h3 = s[...] * (r0[...] + r1[...]) + b3[...]
    h3row = jnp.reshape(h3, (1, BM))
    brow = jnp.reshape(bat[...], (1, BM))
    gid = lax.broadcasted_iota(jnp.int32, (G, BM), 0)
    mask = (brow == gid).astype(jnp.float32)
    part_s = jnp.sum(mask * h3row, axis=1, keepdims=True)
    part_c = jnp.sum(mask, axis=1, keepdims=True)
    part = jnp.concatenate([part_s, part_c], axis=1)

    @pl.when(i == 0)
    def _():
        acc[...] = jnp.zeros_like(acc)

    acc[...] += part

    @pl.when(i == STEPS - 1)
    def _():
        sums = acc[:, 0:1]
        cnts = jnp.maximum(acc[:, 1:2], 1.0)
        out[...] = jax.nn.sigmoid(sums / cnts)


def _tc4(rf, s1, b3r, batchf):
    return pl.pallas_call(
        _tc4_body,
        grid=(STEPS,),
        in_specs=[_fs(16), _fs(16, STEPS), _fs(16), _full((1, 1)), _fs(16)],
        out_specs=_full((G, 1)),
        out_shape=jax.ShapeDtypeStruct((G, 1), jnp.float32),
        scratch_shapes=[pltpu.VMEM((G, 2), jnp.float32)],
    )(rf, rf, s1, b3r, batchf)


def kernel(x, edge_index, batch, W1, b1, W2, b2, W3, b3):
    src = edge_index[0]
    dst = edge_index[1]
    i16 = jnp.eye(16, dtype=jnp.float32)
    spare = N_ACC - N
    trash = N + jnp.arange(EP - E, dtype=jnp.int32) % spare
    srcpad = jnp.arange(EP - E, dtype=jnp.int32) % N
    src_p = jnp.concatenate([src, srcpad])
    dst_p = jnp.concatenate([dst, trash])
    src2d = src_p.reshape(-1, 512)
    dst2d = dst_p.reshape(-1, 512)
    src2c = jnp.concatenate([src_p, src_p + N_ACC]).reshape(-1, 512)
    loop = jnp.arange(N_ACC, dtype=jnp.int32)
    srcw1pad = (jnp.arange(EPS - E - N_ACC, dtype=jnp.int32) % N) * 16
    srcw1 = jnp.concatenate(
        [src * 16, loop * 16, srcw1pad]).reshape(-1, 512)
    trashw1 = N + jnp.arange(EPS - E - N_ACC, dtype=jnp.int32) % spare
    dstw1 = jnp.concatenate([dst, loop, trashw1]).reshape(-1, 512)
    z1 = jnp.zeros((N_ACC,), jnp.float32)
    z8 = jnp.zeros((N_ACC, 8), jnp.float32)
    z16 = jnp.zeros((N_ACC, 16), jnp.float32)
    k8 = jnp.repeat(i16, 8, axis=1)
    k16 = jnp.repeat(i16, 16, axis=1)
    bd1 = jnp.kron(i16, W1.T)
    b1w = jnp.tile(b1, 16)[None, :]
    bd2lo = jnp.kron(i16, W2.T[:, :16])
    bd2hi = jnp.kron(i16, W2.T[:, 16:])
    i8 = jnp.eye(8, dtype=jnp.float32)
    m16lo = jnp.kron(i8, jnp.outer(W3[0, :16], jnp.ones(16, jnp.float32)))
    m16hi = jnp.kron(i8, jnp.outer(W3[0, 16:], jnp.ones(16, jnp.float32)))
    b2lo = jnp.tile(b2[:16], 8)[None, :]
    b2hi = jnp.tile(b2[16:], 8)[None, :]

    degp = _sc_deg(dst2d, z1)
    x_flat = jnp.pad(x, ((0, N_ACC - N), (0, 0))).reshape(8 * RS, 128)
    s1, g0_flat = _tc1(degp.reshape(2 * RS, 128), x_flat, k8)

    q = _sc_w8(g0_flat.reshape(N_ACC, 8), src2d, dst2d, z8)
    glo_f, ghi_f = _tc2(q.reshape(16 * RS, 128), g0_flat, s1, k8, k16,
                        bd1, b1w, bd2lo, bd2hi)

    g2tab = jnp.concatenate([glo_f, ghi_f]).reshape(2 * N_ACC, 16)
    p16 = _sc_w16(g2tab, src2c, dst2d, z16)
    g3rep = _tc3(p16.reshape(32 * RS, 128), glo_f, ghi_f, s1, k16,
                 m16lo, m16hi, b2lo, b2hi)

    r = _sc_w1(g3rep.reshape(16 * N_ACC), srcw1, dstw1, z1)
    batchf = jnp.pad(batch, (0, N_ACC - N),
                     constant_values=G).reshape(RS, 128)
    return _tc4(r.reshape(2 * RS, 128), s1, b3[None, :], batchf)
